# Initial kernel scaffold; baseline (speedup 1.0000x reference)
import functools

import jax
import jax.numpy as jnp
from jax import lax
from jax.experimental import pallas as pl
from jax.experimental.pallas import tpu as pltpu
from jax.experimental.pallas import tpu_sc as plsc

N = 10000
NP = 10240
D = 128
EE = 330000
NTILES = 16
T = 20736
B = 128
NCHUNK = T // B
EEP = NTILES * T
NTS = NP // NTILES
BN = 1024
EPS = 1e-16


def _dense_body(x_ref, w_ref, as_ref, ad_ref, h_ref, hs_ref, hd_ref, gs_ref, gd_ref):
    i = pl.program_id(0)
    h = jnp.dot(x_ref[...], w_ref[...], preferred_element_type=jnp.float32)
    h_ref[...] = h
    hs = jnp.sum(h * as_ref[...], axis=1)
    hd = jnp.sum(h * ad_ref[...], axis=1)
    hs_ref[...] = hs.reshape(1, BN)
    hd_ref[...] = hd.reshape(1, BN)
    bs = jnp.max(hs)
    bd = jnp.max(hd)

    @pl.when(i == 0)
    def _():
        gs_ref[0, 0] = bs
        gd_ref[0, 0] = bd

    @pl.when(i > 0)
    def _():
        gs_ref[0, 0] = jnp.maximum(gs_ref[0, 0], bs)
        gd_ref[0, 0] = jnp.maximum(gd_ref[0, 0], bd)


def _dense1(x, w, a_s, a_d):
    grid = (NP // BN,)
    return pl.pallas_call(
        _dense_body,
        grid=grid,
        in_specs=[
            pl.BlockSpec((BN, D), lambda i: (i, 0)),
            pl.BlockSpec((D, D), lambda i: (0, 0)),
            pl.BlockSpec((1, D), lambda i: (0, 0)),
            pl.BlockSpec((1, D), lambda i: (0, 0)),
        ],
        out_specs=[
            pl.BlockSpec((BN, D), lambda i: (i, 0)),
            pl.BlockSpec((1, BN), lambda i: (0, i)),
            pl.BlockSpec((1, BN), lambda i: (0, i)),
            pl.BlockSpec((1, 1), lambda i: (0, 0)),
            pl.BlockSpec((1, 1), lambda i: (0, 0)),
        ],
        out_shape=[
            jax.ShapeDtypeStruct((NP, D), jnp.float32),
            jax.ShapeDtypeStruct((1, NP), jnp.float32),
            jax.ShapeDtypeStruct((1, NP), jnp.float32),
            jax.ShapeDtypeStruct((1, 1), jnp.float32),
            jax.ShapeDtypeStruct((1, 1), jnp.float32),
        ],
    )(x, w, a_s, a_d)


def _agg_body(srcoff_hbm, dst_hbm, hs_hbm, hd_hbm, g_hbm, hcat_hbm,
              out_hbm, ssum_hbm,
              hs_v, hd_v, g_v, src_v, dst_v, ex_v, rows_v, zs_v,
              out_sh, s_sh, sem):
    c = lax.axis_index("c")
    s = lax.axis_index("s")

    pltpu.sync_copy(hs_hbm, hs_v)
    pltpu.sync_copy(hd_hbm, hd_v)
    pltpu.sync_copy(g_hbm.at[c], g_v)
    g16 = g_v[...]
    coff = (c * NP).astype(jnp.int32)

    z16 = jnp.zeros((16,), jnp.float32)

    def _zrow(r, carry):
        for k in range(D // 16):
            rows_v[r, pl.ds(k * 16, 16)] = z16
        return carry

    lax.fori_loop(0, B, _zrow, 0)
    for k in range(NTS // 16):
        zs_v[pl.ds(k * 16, 16)] = z16
    for j in range(NTS // B):
        pltpu.sync_copy(rows_v, out_sh.at[pl.ds(s * NTS + j * B, B)])
    pltpu.sync_copy(zs_v, s_sh.at[pl.ds(s * NTS, NTS)])
    plsc.subcore_barrier()

    def _chunk(ci, carry):
        base = s * T + ci * B
        pltpu.sync_copy(srcoff_hbm.at[c, pl.ds(base, B)], src_v)
        pltpu.sync_copy(dst_hbm.at[c, pl.ds(base, B)], dst_v)
        pltpu.async_copy(hcat_hbm.at[src_v], rows_v, sem).wait()

        def _ex(j, inner):
            sv = src_v[pl.ds(j * 16, 16)]
            dv = dst_v[pl.ds(j * 16, 16)]
            hsg = plsc.load_gather(hs_v, (sv,))
            hdg = plsc.load_gather(hd_v, (dv + coff,))
            t = hsg + hdg
            e = jnp.where(t >= 0.0, t, 0.2 * t)
            eid = base + j * 16 + lax.iota(jnp.int32, 16)
            e = jnp.where(eid < EE, e, -1e30)
            ex_v[pl.ds(j * 16, 16)] = jnp.exp(e - g16)
            return inner

        lax.fori_loop(0, B // 16, _ex, 0)
        pltpu.sync_copy(ex_v, s_sh.at[dst_v], add=True)

        def _scale(r, inner):
            exr = plsc.load_gather(ex_v, (jnp.broadcast_to(r, (16,)).astype(jnp.int32),))
            for k in range(D // 16):
                rows_v[r, pl.ds(k * 16, 16)] = rows_v[r, pl.ds(k * 16, 16)] * exr
            return inner

        lax.fori_loop(0, B, _scale, 0)
        pltpu.sync_copy(rows_v, out_sh.at[dst_v], add=True)
        return carry

    lax.fori_loop(0, NCHUNK, _chunk, 0)
    plsc.subcore_barrier()

    pltpu.sync_copy(out_sh.at[pl.ds(s * NTS, NTS)],
                    out_hbm.at[c, pl.ds(s * NTS, NTS)])
    pltpu.sync_copy(s_sh.at[pl.ds(s * NTS, NTS)],
                    ssum_hbm.at[c, pl.ds(s * NTS, NTS)])


@functools.partial(
    pl.kernel,
    out_type=[
        jax.ShapeDtypeStruct((2, NP, D), jnp.float32),
        jax.ShapeDtypeStruct((2, NP), jnp.float32),
    ],
    mesh=plsc.VectorSubcoreMesh(core_axis_name="c", subcore_axis_name="s"),
    scratch_types=[
        pltpu.VMEM((2 * NP,), jnp.float32),
        pltpu.VMEM((2 * NP,), jnp.float32),
        pltpu.VMEM((16,), jnp.float32),
        pltpu.VMEM((B,), jnp.int32),
        pltpu.VMEM((B,), jnp.int32),
        pltpu.VMEM((B,), jnp.float32),
        pltpu.VMEM((B, D), jnp.float32),
        pltpu.VMEM((NTS,), jnp.float32),
        pltpu.VMEM_SHARED((NP, D), jnp.float32),
        pltpu.VMEM_SHARED((NP,), jnp.float32),
        pltpu.SemaphoreType.DMA,
    ],
)
def _aggregate(srcoff, dst, hscat, hdcat, g, hcat, out, ssum, *scratch):
    _agg_body(srcoff, dst, hscat, hdcat, g, hcat, out, ssum, *scratch)


def _fuse2_body(oi_ref, si_ref, op_ref, sp_ref, b1i_ref, b1p_ref, fha_ref,
                w2i_ref, w2p_ref, asi_ref, adi_ref, asp_ref, adp_ref,
                h2i_ref, h2p_ref, hs2i_ref, hd2i_ref, hs2p_ref, hd2p_ref,
                gsi_ref, gdi_ref, gsp_ref, gdp_ref):
    i = pl.program_id(0)
    xi = jax.nn.relu(oi_ref[...] / (si_ref[...] + EPS) + b1i_ref[...])
    xp = jax.nn.relu(op_ref[...] / (sp_ref[...] + EPS) + b1p_ref[...])
    fha = fha_ref[...]
    wi = jnp.sum(xi * fha[0:1, :], axis=1, keepdims=True)
    wp = jnp.sum(xp * fha[1:2, :], axis=1, keepdims=True)
    mx = jnp.maximum(wi, wp)
    e0 = jnp.exp(wi - mx)
    e1 = jnp.exp(wp - mx)
    xf = (e0 * xi + e1 * xp) / (e0 + e1)
    h2i = jnp.dot(xf, w2i_ref[...], preferred_element_type=jnp.float32)
    h2p = jnp.dot(xf, w2p_ref[...], preferred_element_type=jnp.float32)
    h2i_ref[...] = h2i
    h2p_ref[...] = h2p
    hs2i = jnp.sum(h2i * asi_ref[...], axis=1)
    hd2i = jnp.sum(h2i * adi_ref[...], axis=1)
    hs2p = jnp.sum(h2p * asp_ref[...], axis=1)
    hd2p = jnp.sum(h2p * adp_ref[...], axis=1)
    hs2i_ref[...] = hs2i.reshape(1, BN)
    hd2i_ref[...] = hd2i.reshape(1, BN)
    hs2p_ref[...] = hs2p.reshape(1, BN)
    hd2p_ref[...] = hd2p.reshape(1, BN)

    @pl.when(i == 0)
    def _():
        gsi_ref[0, 0] = jnp.max(hs2i)
        gdi_ref[0, 0] = jnp.max(hd2i)
        gsp_ref[0, 0] = jnp.max(hs2p)
        gdp_ref[0, 0] = jnp.max(hd2p)

    @pl.when(i > 0)
    def _():
        gsi_ref[0, 0] = jnp.maximum(gsi_ref[0, 0], jnp.max(hs2i))
        gdi_ref[0, 0] = jnp.maximum(gdi_ref[0, 0], jnp.max(hd2i))
        gsp_ref[0, 0] = jnp.maximum(gsp_ref[0, 0], jnp.max(hs2p))
        gdp_ref[0, 0] = jnp.maximum(gdp_ref[0, 0], jnp.max(hd2p))


def _fuse2(oi, si, op, sp, b1i, b1p, fha, w2i, w2p, asi, adi, asp, adp):
    grid = (NP // BN,)
    row = lambda i: (i, 0)
    full = lambda i: (0, 0)
    lane = lambda i: (0, i)
    return pl.pallas_call(
        _fuse2_body,
        grid=grid,
        in_specs=[
            pl.BlockSpec((BN, D), row),
            pl.BlockSpec((BN, 1), row),
            pl.BlockSpec((BN, D), row),
            pl.BlockSpec((BN, 1), row),
            pl.BlockSpec((1, D), full),
            pl.BlockSpec((1, D), full),
            pl.BlockSpec((2, D), full),
            pl.BlockSpec((D, D), full),
            pl.BlockSpec((D, D), full),
            pl.BlockSpec((1, D), full),
            pl.BlockSpec((1, D), full),
            pl.BlockSpec((1, D), full),
            pl.BlockSpec((1, D), full),
        ],
        out_specs=[
            pl.BlockSpec((BN, D), row),
            pl.BlockSpec((BN, D), row),
            pl.BlockSpec((1, BN), lane),
            pl.BlockSpec((1, BN), lane),
            pl.BlockSpec((1, BN), lane),
            pl.BlockSpec((1, BN), lane),
            pl.BlockSpec((1, 1), full),
            pl.BlockSpec((1, 1), full),
            pl.BlockSpec((1, 1), full),
            pl.BlockSpec((1, 1), full),
        ],
        out_shape=[
            jax.ShapeDtypeStruct((NP, D), jnp.float32),
            jax.ShapeDtypeStruct((NP, D), jnp.float32),
            jax.ShapeDtypeStruct((1, NP), jnp.float32),
            jax.ShapeDtypeStruct((1, NP), jnp.float32),
            jax.ShapeDtypeStruct((1, NP), jnp.float32),
            jax.ShapeDtypeStruct((1, NP), jnp.float32),
            jax.ShapeDtypeStruct((1, 1), jnp.float32),
            jax.ShapeDtypeStruct((1, 1), jnp.float32),
            jax.ShapeDtypeStruct((1, 1), jnp.float32),
            jax.ShapeDtypeStruct((1, 1), jnp.float32),
        ],
    )(oi, si, op, sp, b1i, b1p, fha, w2i, w2p, asi, adi, asp, adp)


def _head_body(oi_ref, si_ref, op_ref, sp_ref, b2i_ref, b2p_ref, aw_ref,
               mw_ref, mb_ref, y_ref):
    xi = oi_ref[...] / (si_ref[...] + EPS) + b2i_ref[...]
    xp = op_ref[...] / (sp_ref[...] + EPS) + b2p_ref[...]
    aw = aw_ref[...]
    wi = jnp.sum(xi * aw[0:1, :], axis=1, keepdims=True)
    wp = jnp.sum(xp * aw[1:2, :], axis=1, keepdims=True)
    mx = jnp.maximum(wi, wp)
    e0 = jnp.exp(wi - mx)
    e1 = jnp.exp(wp - mx)
    x = (e0 * xi + e1 * xp) / (e0 + e1)
    y_ref[...] = jnp.dot(x, mw_ref[...], preferred_element_type=jnp.float32) + mb_ref[...]


def _head(oi, si, op, sp, b2i, b2p, aw, mw, mb):
    grid = (NP // BN,)
    row = lambda i: (i, 0)
    full = lambda i: (0, 0)
    return pl.pallas_call(
        _head_body,
        grid=grid,
        in_specs=[
            pl.BlockSpec((BN, D), row),
            pl.BlockSpec((BN, 1), row),
            pl.BlockSpec((BN, D), row),
            pl.BlockSpec((BN, 1), row),
            pl.BlockSpec((1, D), full),
            pl.BlockSpec((1, D), full),
            pl.BlockSpec((2, D), full),
            pl.BlockSpec((D, 1), full),
            pl.BlockSpec((1, 1), full),
        ],
        out_specs=pl.BlockSpec((BN, 1), row),
        out_shape=jax.ShapeDtypeStruct((NP, 1), jnp.float32),
    )(oi, si, op, sp, b2i, b2p, aw, mw, mb)


def kernel(x_industry, edge_index_industry, x_pos_corr, edge_index_pos,
           W1i, a1i_s, a1i_d, b1i, W2i, a2i_s, a2i_d, b2i,
           W1p, a1p_s, a1p_d, b1p, W2p, a2p_s, a2p_d, b2p,
           fha, aw, mlp_W, mlp_b):
    f32 = jnp.float32
    loop = jnp.arange(N, dtype=jnp.int32)
    zpad = jnp.zeros((EEP - EE,), jnp.int32)

    def edges(ei, off):
        src = jnp.concatenate([ei[0].astype(jnp.int32), loop, zpad]) + off
        dst = jnp.concatenate([ei[1].astype(jnp.int32), loop, zpad])
        return src, dst

    src_i, dst_i = edges(edge_index_industry, 0)
    src_p, dst_p = edges(edge_index_pos, NP)
    srcoff = jnp.stack([src_i, src_p])
    dst2 = jnp.stack([dst_i, dst_p])

    xpad = jnp.zeros((NP - N, D), f32)
    xi_in = jnp.concatenate([x_industry, xpad], axis=0)
    xp_in = jnp.concatenate([x_pos_corr, xpad], axis=0)

    r1 = lambda v: v.reshape(1, D)

    h1i, hs1i, hd1i, gs1i, gd1i = _dense1(xi_in, W1i, r1(a1i_s), r1(a1i_d))
    h1p, hs1p, hd1p, gs1p, gd1p = _dense1(xp_in, W1p, r1(a1p_s), r1(a1p_d))

    def gbound(gs, gd):
        t = gs[0, 0] + gd[0, 0]
        return jnp.broadcast_to(jnp.where(t >= 0.0, t, 0.2 * t), (16,))

    g1 = jnp.stack([gbound(gs1i, gd1i), gbound(gs1p, gd1p)])
    hscat1 = jnp.concatenate([hs1i[0], hs1p[0]])
    hdcat1 = jnp.concatenate([hd1i[0], hd1p[0]])
    hcat1 = jnp.concatenate([h1i, h1p], axis=0)

    out1, ssum1 = _aggregate(srcoff, dst2, hscat1, hdcat1, g1, hcat1)

    c1 = lambda v: v.reshape(NP, 1)
    h2i, h2p, hs2i, hd2i, hs2p, hd2p, gs2i, gd2i, gs2p, gd2p = _fuse2(
        out1[0], c1(ssum1[0]), out1[1], c1(ssum1[1]),
        r1(b1i), r1(b1p), fha, W2i, W2p,
        r1(a2i_s), r1(a2i_d), r1(a2p_s), r1(a2p_d))

    g2 = jnp.stack([gbound(gs2i, gd2i), gbound(gs2p, gd2p)])
    hscat2 = jnp.concatenate([hs2i[0], hs2p[0]])
    hdcat2 = jnp.concatenate([hd2i[0], hd2p[0]])
    hcat2 = jnp.concatenate([h2i, h2p], axis=0)

    out2, ssum2 = _aggregate(srcoff, dst2, hscat2, hdcat2, g2, hcat2)

    y = _head(out2[0], c1(ssum2[0]), out2[1], c1(ssum2[1]),
              r1(b2i), r1(b2p), aw, mlp_W, mlp_b.reshape(1, 1))
    return y[:N]


# speedup vs baseline: 22.5924x; 22.5924x over previous
import functools

import jax
import jax.numpy as jnp
from jax import lax
from jax.experimental import pallas as pl
from jax.experimental.pallas import tpu as pltpu
from jax.experimental.pallas import tpu_sc as plsc

N = 10000
NP = 10240
D = 128
EE = 330000
NTILES = 16
T = 20736
B = 128
NCHUNK = T // B
EEP = NTILES * T
NTS = NP // NTILES
BN = 1024
EPS = 1e-16


def _dense_body(x_ref, w_ref, as_ref, ad_ref, h_ref, hs_ref, hd_ref, gs_ref, gd_ref):
    i = pl.program_id(0)
    h = jnp.dot(x_ref[...], w_ref[...], preferred_element_type=jnp.float32)
    h_ref[...] = h
    hs = jnp.sum(h * as_ref[...], axis=1)
    hd = jnp.sum(h * ad_ref[...], axis=1)
    hs_ref[...] = hs.reshape(1, BN)
    hd_ref[...] = hd.reshape(1, BN)
    bs = jnp.max(hs)
    bd = jnp.max(hd)

    @pl.when(i == 0)
    def _():
        gs_ref[0, 0] = bs
        gd_ref[0, 0] = bd

    @pl.when(i > 0)
    def _():
        gs_ref[0, 0] = jnp.maximum(gs_ref[0, 0], bs)
        gd_ref[0, 0] = jnp.maximum(gd_ref[0, 0], bd)


def _dense1(x, w, a_s, a_d):
    grid = (NP // BN,)
    return pl.pallas_call(
        _dense_body,
        grid=grid,
        in_specs=[
            pl.BlockSpec((BN, D), lambda i: (i, 0)),
            pl.BlockSpec((D, D), lambda i: (0, 0)),
            pl.BlockSpec((1, D), lambda i: (0, 0)),
            pl.BlockSpec((1, D), lambda i: (0, 0)),
        ],
        out_specs=[
            pl.BlockSpec((BN, D), lambda i: (i, 0)),
            pl.BlockSpec((1, BN), lambda i: (0, i)),
            pl.BlockSpec((1, BN), lambda i: (0, i)),
            pl.BlockSpec(memory_space=pltpu.SMEM),
            pl.BlockSpec(memory_space=pltpu.SMEM),
        ],
        out_shape=[
            jax.ShapeDtypeStruct((NP, D), jnp.float32),
            jax.ShapeDtypeStruct((1, NP), jnp.float32),
            jax.ShapeDtypeStruct((1, NP), jnp.float32),
            jax.ShapeDtypeStruct((1, 1), jnp.float32),
            jax.ShapeDtypeStruct((1, 1), jnp.float32),
        ],
    )(x, w, a_s, a_d)


def _agg_body(srcoff_hbm, dst_hbm, hs_hbm, hd_hbm, g_hbm, hcat_hbm,
              out_hbm, ssum_hbm,
              g_v, src_v, dst_v, dstoff_v, hsg_v, hdg_v, ex_v, rows_v, zs_v,
              out_sh, s_sh, hs_sh, hd_sh, sem):
    c = lax.axis_index("c")
    s = lax.axis_index("s")

    pltpu.sync_copy(g_hbm.at[c], g_v)
    g16 = g_v[...]
    coff = (c * NP).astype(jnp.int32)

    @pl.when(s == 0)
    def _():
        pltpu.sync_copy(hs_hbm, hs_sh)
        pltpu.sync_copy(hd_hbm, hd_sh)

    z16 = jnp.zeros((16,), jnp.float32)

    def _zrow(r, carry):
        for k in range(D // 16):
            rows_v[r, pl.ds(k * 16, 16)] = z16
        return carry

    lax.fori_loop(0, B, _zrow, 0)
    for k in range(NTS // 16):
        zs_v[pl.ds(k * 16, 16)] = z16
    for j in range(NTS // B):
        pltpu.sync_copy(rows_v, out_sh.at[pl.ds(s * NTS + j * B, B)])
    pltpu.sync_copy(zs_v, s_sh.at[pl.ds(s * NTS, NTS)])
    plsc.subcore_barrier()

    def _chunk(ci, carry):
        base = s * T + ci * B
        pltpu.sync_copy(srcoff_hbm.at[c, pl.ds(base, B)], src_v)
        pltpu.sync_copy(dst_hbm.at[c, pl.ds(base, B)], dst_v)
        pltpu.async_copy(hcat_hbm.at[src_v], rows_v, sem).wait()

        def _off(j, inner):
            dstoff_v[pl.ds(j * 16, 16)] = dst_v[pl.ds(j * 16, 16)] + coff
            return inner

        lax.fori_loop(0, B // 16, _off, 0)
        pltpu.async_copy(hs_sh.at[src_v], hsg_v, sem).wait()
        pltpu.async_copy(hd_sh.at[dstoff_v], hdg_v, sem).wait()

        def _ex(j, inner):
            t = hsg_v[pl.ds(j * 16, 16)] + hdg_v[pl.ds(j * 16, 16)]
            e = jnp.where(t >= 0.0, t, 0.2 * t)
            eid = base + j * 16 + lax.iota(jnp.int32, 16)
            e = jnp.where(eid < EE, e, -1e30)
            ex_v[pl.ds(j * 16, 16)] = jnp.exp(e - g16)
            return inner

        lax.fori_loop(0, B // 16, _ex, 0)
        pltpu.sync_copy(ex_v, s_sh.at[dst_v], add=True)

        def _scale(r, inner):
            exr = plsc.load_gather(ex_v, (jnp.broadcast_to(r, (16,)).astype(jnp.int32),))
            for k in range(D // 16):
                rows_v[r, pl.ds(k * 16, 16)] = rows_v[r, pl.ds(k * 16, 16)] * exr
            return inner

        lax.fori_loop(0, B, _scale, 0)
        pltpu.sync_copy(rows_v, out_sh.at[dst_v], add=True)
        return carry

    lax.fori_loop(0, NCHUNK, _chunk, 0)
    plsc.subcore_barrier()

    pltpu.sync_copy(out_sh.at[pl.ds(s * NTS, NTS)],
                    out_hbm.at[c, pl.ds(s * NTS, NTS)])
    pltpu.sync_copy(s_sh.at[pl.ds(s * NTS, NTS)],
                    ssum_hbm.at[c, pl.ds(s * NTS, NTS)])


@functools.partial(
    pl.kernel,
    out_type=[
        jax.ShapeDtypeStruct((2, NP, D), jnp.float32),
        jax.ShapeDtypeStruct((2, NP), jnp.float32),
    ],
    mesh=plsc.VectorSubcoreMesh(core_axis_name="c", subcore_axis_name="s"),
    compiler_params=pltpu.CompilerParams(needs_layout_passes=False),
    scratch_types=[
        pltpu.VMEM((16,), jnp.float32),
        pltpu.VMEM((B,), jnp.int32),
        pltpu.VMEM((B,), jnp.int32),
        pltpu.VMEM((B,), jnp.int32),
        pltpu.VMEM((B,), jnp.float32),
        pltpu.VMEM((B,), jnp.float32),
        pltpu.VMEM((B,), jnp.float32),
        pltpu.VMEM((B, D), jnp.float32),
        pltpu.VMEM((NTS,), jnp.float32),
        pltpu.VMEM_SHARED((NP, D), jnp.float32),
        pltpu.VMEM_SHARED((NP,), jnp.float32),
        pltpu.VMEM_SHARED((2 * NP,), jnp.float32),
        pltpu.VMEM_SHARED((2 * NP,), jnp.float32),
        pltpu.SemaphoreType.DMA,
    ],
)
def _aggregate(srcoff, dst, hscat, hdcat, g, hcat, out, ssum, *scratch):
    _agg_body(srcoff, dst, hscat, hdcat, g, hcat, out, ssum, *scratch)


def _fuse2_body(oi_ref, si_ref, op_ref, sp_ref, b1i_ref, b1p_ref, fha_ref,
                w2i_ref, w2p_ref, asi_ref, adi_ref, asp_ref, adp_ref,
                h2i_ref, h2p_ref, hs2i_ref, hd2i_ref, hs2p_ref, hd2p_ref,
                gsi_ref, gdi_ref, gsp_ref, gdp_ref):
    i = pl.program_id(0)
    xi = jax.nn.relu(oi_ref[...] / (si_ref[...] + EPS) + b1i_ref[...])
    xp = jax.nn.relu(op_ref[...] / (sp_ref[...] + EPS) + b1p_ref[...])
    fha = fha_ref[...]
    wi = jnp.sum(xi * fha[0:1, :], axis=1, keepdims=True)
    wp = jnp.sum(xp * fha[1:2, :], axis=1, keepdims=True)
    mx = jnp.maximum(wi, wp)
    e0 = jnp.exp(wi - mx)
    e1 = jnp.exp(wp - mx)
    xf = (e0 * xi + e1 * xp) / (e0 + e1)
    h2i = jnp.dot(xf, w2i_ref[...], preferred_element_type=jnp.float32)
    h2p = jnp.dot(xf, w2p_ref[...], preferred_element_type=jnp.float32)
    h2i_ref[...] = h2i
    h2p_ref[...] = h2p
    hs2i = jnp.sum(h2i * asi_ref[...], axis=1)
    hd2i = jnp.sum(h2i * adi_ref[...], axis=1)
    hs2p = jnp.sum(h2p * asp_ref[...], axis=1)
    hd2p = jnp.sum(h2p * adp_ref[...], axis=1)
    hs2i_ref[...] = hs2i.reshape(1, BN)
    hd2i_ref[...] = hd2i.reshape(1, BN)
    hs2p_ref[...] = hs2p.reshape(1, BN)
    hd2p_ref[...] = hd2p.reshape(1, BN)

    @pl.when(i == 0)
    def _():
        gsi_ref[0, 0] = jnp.max(hs2i)
        gdi_ref[0, 0] = jnp.max(hd2i)
        gsp_ref[0, 0] = jnp.max(hs2p)
        gdp_ref[0, 0] = jnp.max(hd2p)

    @pl.when(i > 0)
    def _():
        gsi_ref[0, 0] = jnp.maximum(gsi_ref[0, 0], jnp.max(hs2i))
        gdi_ref[0, 0] = jnp.maximum(gdi_ref[0, 0], jnp.max(hd2i))
        gsp_ref[0, 0] = jnp.maximum(gsp_ref[0, 0], jnp.max(hs2p))
        gdp_ref[0, 0] = jnp.maximum(gdp_ref[0, 0], jnp.max(hd2p))


def _fuse2(oi, si, op, sp, b1i, b1p, fha, w2i, w2p, asi, adi, asp, adp):
    grid = (NP // BN,)
    row = lambda i: (i, 0)
    full = lambda i: (0, 0)
    lane = lambda i: (0, i)
    return pl.pallas_call(
        _fuse2_body,
        grid=grid,
        in_specs=[
            pl.BlockSpec((BN, D), row),
            pl.BlockSpec((BN, 1), row),
            pl.BlockSpec((BN, D), row),
            pl.BlockSpec((BN, 1), row),
            pl.BlockSpec((1, D), full),
            pl.BlockSpec((1, D), full),
            pl.BlockSpec((2, D), full),
            pl.BlockSpec((D, D), full),
            pl.BlockSpec((D, D), full),
            pl.BlockSpec((1, D), full),
            pl.BlockSpec((1, D), full),
            pl.BlockSpec((1, D), full),
            pl.BlockSpec((1, D), full),
        ],
        out_specs=[
            pl.BlockSpec((BN, D), row),
            pl.BlockSpec((BN, D), row),
            pl.BlockSpec((1, BN), lane),
            pl.BlockSpec((1, BN), lane),
            pl.BlockSpec((1, BN), lane),
            pl.BlockSpec((1, BN), lane),
            pl.BlockSpec(memory_space=pltpu.SMEM),
            pl.BlockSpec(memory_space=pltpu.SMEM),
            pl.BlockSpec(memory_space=pltpu.SMEM),
            pl.BlockSpec(memory_space=pltpu.SMEM),
        ],
        out_shape=[
            jax.ShapeDtypeStruct((NP, D), jnp.float32),
            jax.ShapeDtypeStruct((NP, D), jnp.float32),
            jax.ShapeDtypeStruct((1, NP), jnp.float32),
            jax.ShapeDtypeStruct((1, NP), jnp.float32),
            jax.ShapeDtypeStruct((1, NP), jnp.float32),
            jax.ShapeDtypeStruct((1, NP), jnp.float32),
            jax.ShapeDtypeStruct((1, 1), jnp.float32),
            jax.ShapeDtypeStruct((1, 1), jnp.float32),
            jax.ShapeDtypeStruct((1, 1), jnp.float32),
            jax.ShapeDtypeStruct((1, 1), jnp.float32),
        ],
    )(oi, si, op, sp, b1i, b1p, fha, w2i, w2p, asi, adi, asp, adp)


def _head_body(oi_ref, si_ref, op_ref, sp_ref, b2i_ref, b2p_ref, aw_ref,
               mw_ref, mb_ref, y_ref):
    xi = oi_ref[...] / (si_ref[...] + EPS) + b2i_ref[...]
    xp = op_ref[...] / (sp_ref[...] + EPS) + b2p_ref[...]
    aw = aw_ref[...]
    wi = jnp.sum(xi * aw[0:1, :], axis=1, keepdims=True)
    wp = jnp.sum(xp * aw[1:2, :], axis=1, keepdims=True)
    mx = jnp.maximum(wi, wp)
    e0 = jnp.exp(wi - mx)
    e1 = jnp.exp(wp - mx)
    x = (e0 * xi + e1 * xp) / (e0 + e1)
    y_ref[...] = jnp.dot(x, mw_ref[...], preferred_element_type=jnp.float32) + mb_ref[...]


def _head(oi, si, op, sp, b2i, b2p, aw, mw, mb):
    grid = (NP // BN,)
    row = lambda i: (i, 0)
    full = lambda i: (0, 0)
    return pl.pallas_call(
        _head_body,
        grid=grid,
        in_specs=[
            pl.BlockSpec((BN, D), row),
            pl.BlockSpec((BN, 1), row),
            pl.BlockSpec((BN, D), row),
            pl.BlockSpec((BN, 1), row),
            pl.BlockSpec((1, D), full),
            pl.BlockSpec((1, D), full),
            pl.BlockSpec((2, D), full),
            pl.BlockSpec((D, 1), full),
            pl.BlockSpec((1, 1), full),
        ],
        out_specs=pl.BlockSpec((BN, 1), row),
        out_shape=jax.ShapeDtypeStruct((NP, 1), jnp.float32),
    )(oi, si, op, sp, b2i, b2p, aw, mw, mb)


def kernel(x_industry, edge_index_industry, x_pos_corr, edge_index_pos,
           W1i, a1i_s, a1i_d, b1i, W2i, a2i_s, a2i_d, b2i,
           W1p, a1p_s, a1p_d, b1p, W2p, a2p_s, a2p_d, b2p,
           fha, aw, mlp_W, mlp_b):
    f32 = jnp.float32
    loop = jnp.arange(N, dtype=jnp.int32)
    zpad = jnp.zeros((EEP - EE,), jnp.int32)

    def edges(ei, off):
        src = jnp.concatenate([ei[0].astype(jnp.int32), loop, zpad]) + off
        dst = jnp.concatenate([ei[1].astype(jnp.int32), loop, zpad])
        return src, dst

    src_i, dst_i = edges(edge_index_industry, 0)
    src_p, dst_p = edges(edge_index_pos, NP)
    srcoff = jnp.stack([src_i, src_p])
    dst2 = jnp.stack([dst_i, dst_p])

    xpad = jnp.zeros((NP - N, D), f32)
    xi_in = jnp.concatenate([x_industry, xpad], axis=0)
    xp_in = jnp.concatenate([x_pos_corr, xpad], axis=0)

    r1 = lambda v: v.reshape(1, D)

    h1i, hs1i, hd1i, gs1i, gd1i = _dense1(xi_in, W1i, r1(a1i_s), r1(a1i_d))
    h1p, hs1p, hd1p, gs1p, gd1p = _dense1(xp_in, W1p, r1(a1p_s), r1(a1p_d))

    def gbound(gs, gd):
        t = gs[0, 0] + gd[0, 0]
        return jnp.broadcast_to(jnp.where(t >= 0.0, t, 0.2 * t), (16,))

    g1 = jnp.stack([gbound(gs1i, gd1i), gbound(gs1p, gd1p)])
    hscat1 = jnp.concatenate([hs1i[0], hs1p[0]])
    hdcat1 = jnp.concatenate([hd1i[0], hd1p[0]])
    hcat1 = jnp.concatenate([h1i, h1p], axis=0)

    out1, ssum1 = _aggregate(srcoff, dst2, hscat1, hdcat1, g1, hcat1)

    c1 = lambda v: v.reshape(NP, 1)
    h2i, h2p, hs2i, hd2i, hs2p, hd2p, gs2i, gd2i, gs2p, gd2p = _fuse2(
        out1[0], c1(ssum1[0]), out1[1], c1(ssum1[1]),
        r1(b1i), r1(b1p), fha, W2i, W2p,
        r1(a2i_s), r1(a2i_d), r1(a2p_s), r1(a2p_d))

    g2 = jnp.stack([gbound(gs2i, gd2i), gbound(gs2p, gd2p)])
    hscat2 = jnp.concatenate([hs2i[0], hs2p[0]])
    hdcat2 = jnp.concatenate([hd2i[0], hd2p[0]])
    hcat2 = jnp.concatenate([h2i, h2p], axis=0)

    out2, ssum2 = _aggregate(srcoff, dst2, hscat2, hdcat2, g2, hcat2)

    y = _head(out2[0], c1(ssum2[0]), out2[1], c1(ssum2[1]),
              r1(b2i), r1(b2p), aw, mlp_W, mlp_b.reshape(1, 1))
    return y[:N]


# speedup vs baseline: 30.5239x; 1.3511x over previous
import functools

import jax
import jax.numpy as jnp
from jax import lax
from jax.experimental import pallas as pl
from jax.experimental.pallas import tpu as pltpu
from jax.experimental.pallas import tpu_sc as plsc

N = 10000
NP = 10240
D = 128
EE = 330000
NTILES = 16
T = 20736
B = 128
NCHUNK = T // B
EEP = NTILES * T
NTS = NP // NTILES
BN = 1024
EPS = 1e-16


def _dense_body(x_ref, w_ref, as_ref, ad_ref, h_ref, hs_ref, hd_ref, gs_ref, gd_ref):
    i = pl.program_id(0)
    h = jnp.dot(x_ref[...], w_ref[...], preferred_element_type=jnp.float32)
    h_ref[...] = h
    hs = jnp.sum(h * as_ref[...], axis=1)
    hd = jnp.sum(h * ad_ref[...], axis=1)
    hs_ref[...] = hs.reshape(1, BN)
    hd_ref[...] = hd.reshape(1, BN)
    bs = jnp.max(hs)
    bd = jnp.max(hd)

    @pl.when(i == 0)
    def _():
        gs_ref[0, 0] = bs
        gd_ref[0, 0] = bd

    @pl.when(i > 0)
    def _():
        gs_ref[0, 0] = jnp.maximum(gs_ref[0, 0], bs)
        gd_ref[0, 0] = jnp.maximum(gd_ref[0, 0], bd)


def _dense1(x, w, a_s, a_d):
    grid = (NP // BN,)
    return pl.pallas_call(
        _dense_body,
        grid=grid,
        in_specs=[
            pl.BlockSpec((BN, D), lambda i: (i, 0)),
            pl.BlockSpec((D, D), lambda i: (0, 0)),
            pl.BlockSpec((1, D), lambda i: (0, 0)),
            pl.BlockSpec((1, D), lambda i: (0, 0)),
        ],
        out_specs=[
            pl.BlockSpec((BN, D), lambda i: (i, 0)),
            pl.BlockSpec((1, BN), lambda i: (0, i)),
            pl.BlockSpec((1, BN), lambda i: (0, i)),
            pl.BlockSpec(memory_space=pltpu.SMEM),
            pl.BlockSpec(memory_space=pltpu.SMEM),
        ],
        out_shape=[
            jax.ShapeDtypeStruct((NP, D), jnp.float32),
            jax.ShapeDtypeStruct((1, NP), jnp.float32),
            jax.ShapeDtypeStruct((1, NP), jnp.float32),
            jax.ShapeDtypeStruct((1, 1), jnp.float32),
            jax.ShapeDtypeStruct((1, 1), jnp.float32),
        ],
    )(x, w, a_s, a_d)


def _agg_body(srcoff_hbm, dst_hbm, hs_hbm, hd_hbm, g_hbm, hcat_hbm,
              out_hbm, ssum_hbm,
              g_v, zs_v, bufs, out_sh, s_sh, hs_sh, hd_sh):
    c = lax.axis_index("c")
    s = lax.axis_index("s")

    pltpu.sync_copy(g_hbm.at[pl.ds(c * 16, 16)], g_v)
    g16 = g_v[...]
    coff = (c * NP).astype(jnp.int32)

    @pl.when(s == 0)
    def _():
        pltpu.sync_copy(hs_hbm, hs_sh)
        pltpu.sync_copy(hd_hbm, hd_sh)

    z16 = jnp.zeros((16,), jnp.float32)
    rows0 = bufs[0][6]

    def _zrow(r, carry):
        for k in range(D // 16):
            rows0[r, pl.ds(k * 16, 16)] = z16
        return carry

    lax.fori_loop(0, B, _zrow, 0)
    for k in range(NTS // 16):
        zs_v[pl.ds(k * 16, 16)] = z16
    for j in range(NTS // B):
        pltpu.sync_copy(rows0, out_sh.at[pl.ds(s * NTS + j * B, B)])
    rem = NTS - (NTS // B) * B
    if rem:
        pltpu.sync_copy(rows0.at[pl.ds(0, rem)],
                        out_sh.at[pl.ds(s * NTS + (NTS // B) * B, rem)])
    pltpu.sync_copy(zs_v, s_sh.at[pl.ds(s * NTS, NTS)])
    plsc.subcore_barrier()

    def _prefetch(ci, buf):
        src_b, dst_b, doff_b, hsg_b, hdg_b, ex_b, rows_b, semg = buf
        base = s * T + ci * B
        pltpu.sync_copy(srcoff_hbm.at[pl.ds(c * EEP + base, B)], src_b)
        pltpu.sync_copy(dst_hbm.at[pl.ds(c * EEP + base, B)], dst_b)
        pltpu.async_copy(hcat_hbm.at[src_b], rows_b, semg)

    def _process(ci, buf):
        src_b, dst_b, doff_b, hsg_b, hdg_b, ex_b, rows_b, semg = buf
        base = s * T + ci * B
        for j in range(B // 16):
            doff_b[pl.ds(j * 16, 16)] = dst_b[pl.ds(j * 16, 16)] + coff
        pltpu.sync_copy(hs_sh.at[src_b], hsg_b)
        pltpu.sync_copy(hd_sh.at[doff_b], hdg_b)
        pltpu.make_async_copy(hcat_hbm.at[src_b], rows_b, semg).wait()
        for j in range(B // 16):
            t = hsg_b[pl.ds(j * 16, 16)] + hdg_b[pl.ds(j * 16, 16)]
            e = jnp.where(t >= 0.0, t, 0.2 * t)
            eid = base + j * 16 + lax.iota(jnp.int32, 16)
            e = jnp.where(eid < EE, e, -1e30)
            ex_b[pl.ds(j * 16, 16)] = jnp.exp(e - g16)
        pltpu.sync_copy(ex_b, s_sh.at[dst_b], add=True)

        def _scale(r, inner):
            for u in range(2):
                exr = plsc.load_gather(
                    ex_b, (jnp.broadcast_to(2 * r + u, (16,)).astype(jnp.int32),))
                for k in range(D // 16):
                    rows_b[2 * r + u, pl.ds(k * 16, 16)] = (
                        rows_b[2 * r + u, pl.ds(k * 16, 16)] * exr)
            return inner

        lax.fori_loop(0, B // 2, _scale, 0)
        pltpu.sync_copy(rows_b, out_sh.at[dst_b], add=True)

    _prefetch(0, bufs[0])

    def _outer(oi, carry):
        _prefetch(2 * oi + 1, bufs[1])
        _process(2 * oi, bufs[0])
        _prefetch(2 * oi + 2, bufs[0])
        _process(2 * oi + 1, bufs[1])
        return carry

    lax.fori_loop(0, NCHUNK // 2 - 1, _outer, 0)
    _prefetch(NCHUNK - 1, bufs[1])
    _process(NCHUNK - 2, bufs[0])
    _process(NCHUNK - 1, bufs[1])
    plsc.subcore_barrier()

    pltpu.sync_copy(out_sh.at[pl.ds(s * NTS, NTS)],
                    out_hbm.at[c, pl.ds(s * NTS, NTS)])
    pltpu.sync_copy(s_sh.at[pl.ds(s * NTS, NTS)],
                    ssum_hbm.at[pl.ds(c * NP + s * NTS, NTS)])


def _buf_types():
    t = []
    for _ in range(2):
        t.append((
            pltpu.VMEM((B,), jnp.int32),
            pltpu.VMEM((B,), jnp.int32),
            pltpu.VMEM((B,), jnp.int32),
            pltpu.VMEM((B,), jnp.float32),
            pltpu.VMEM((B,), jnp.float32),
            pltpu.VMEM((B,), jnp.float32),
            pltpu.VMEM((B, D), jnp.float32),
            pltpu.SemaphoreType.DMA,
        ))
    return tuple(t)


@functools.partial(
    pl.kernel,
    out_type=[
        jax.ShapeDtypeStruct((2, NP, D), jnp.float32),
        jax.ShapeDtypeStruct((2 * NP,), jnp.float32),
    ],
    mesh=plsc.VectorSubcoreMesh(core_axis_name="c", subcore_axis_name="s"),
    compiler_params=pltpu.CompilerParams(needs_layout_passes=False),
    scratch_types=[
        pltpu.VMEM((16,), jnp.float32),
        pltpu.VMEM((NTS,), jnp.float32),
        _buf_types(),
        pltpu.VMEM_SHARED((NP, D), jnp.float32),
        pltpu.VMEM_SHARED((NP,), jnp.float32),
        pltpu.VMEM_SHARED((2 * NP,), jnp.float32),
        pltpu.VMEM_SHARED((2 * NP,), jnp.float32),
    ],
)
def _aggregate(srcoff, dst, hscat, hdcat, g, hcat, out, ssum, *scratch):
    _agg_body(srcoff, dst, hscat, hdcat, g, hcat, out, ssum, *scratch)


def _fuse2_body(oi_ref, si_ref, op_ref, sp_ref, b1i_ref, b1p_ref, fha_ref,
                w2i_ref, w2p_ref, asi_ref, adi_ref, asp_ref, adp_ref,
                h2i_ref, h2p_ref, hs2i_ref, hd2i_ref, hs2p_ref, hd2p_ref,
                gsi_ref, gdi_ref, gsp_ref, gdp_ref):
    i = pl.program_id(0)
    xi = jax.nn.relu(oi_ref[...] / (si_ref[...] + EPS) + b1i_ref[...])
    xp = jax.nn.relu(op_ref[...] / (sp_ref[...] + EPS) + b1p_ref[...])
    fha = fha_ref[...]
    wi = jnp.sum(xi * fha[0:1, :], axis=1, keepdims=True)
    wp = jnp.sum(xp * fha[1:2, :], axis=1, keepdims=True)
    mx = jnp.maximum(wi, wp)
    e0 = jnp.exp(wi - mx)
    e1 = jnp.exp(wp - mx)
    xf = (e0 * xi + e1 * xp) / (e0 + e1)
    h2i = jnp.dot(xf, w2i_ref[...], preferred_element_type=jnp.float32)
    h2p = jnp.dot(xf, w2p_ref[...], preferred_element_type=jnp.float32)
    h2i_ref[...] = h2i
    h2p_ref[...] = h2p
    hs2i = jnp.sum(h2i * asi_ref[...], axis=1)
    hd2i = jnp.sum(h2i * adi_ref[...], axis=1)
    hs2p = jnp.sum(h2p * asp_ref[...], axis=1)
    hd2p = jnp.sum(h2p * adp_ref[...], axis=1)
    hs2i_ref[...] = hs2i.reshape(1, BN)
    hd2i_ref[...] = hd2i.reshape(1, BN)
    hs2p_ref[...] = hs2p.reshape(1, BN)
    hd2p_ref[...] = hd2p.reshape(1, BN)

    @pl.when(i == 0)
    def _():
        gsi_ref[0, 0] = jnp.max(hs2i)
        gdi_ref[0, 0] = jnp.max(hd2i)
        gsp_ref[0, 0] = jnp.max(hs2p)
        gdp_ref[0, 0] = jnp.max(hd2p)

    @pl.when(i > 0)
    def _():
        gsi_ref[0, 0] = jnp.maximum(gsi_ref[0, 0], jnp.max(hs2i))
        gdi_ref[0, 0] = jnp.maximum(gdi_ref[0, 0], jnp.max(hd2i))
        gsp_ref[0, 0] = jnp.maximum(gsp_ref[0, 0], jnp.max(hs2p))
        gdp_ref[0, 0] = jnp.maximum(gdp_ref[0, 0], jnp.max(hd2p))


def _fuse2(oi, si, op, sp, b1i, b1p, fha, w2i, w2p, asi, adi, asp, adp):
    grid = (NP // BN,)
    row = lambda i: (i, 0)
    full = lambda i: (0, 0)
    lane = lambda i: (0, i)
    return pl.pallas_call(
        _fuse2_body,
        grid=grid,
        in_specs=[
            pl.BlockSpec((BN, D), row),
            pl.BlockSpec((BN, 1), row),
            pl.BlockSpec((BN, D), row),
            pl.BlockSpec((BN, 1), row),
            pl.BlockSpec((1, D), full),
            pl.BlockSpec((1, D), full),
            pl.BlockSpec((2, D), full),
            pl.BlockSpec((D, D), full),
            pl.BlockSpec((D, D), full),
            pl.BlockSpec((1, D), full),
            pl.BlockSpec((1, D), full),
            pl.BlockSpec((1, D), full),
            pl.BlockSpec((1, D), full),
        ],
        out_specs=[
            pl.BlockSpec((BN, D), row),
            pl.BlockSpec((BN, D), row),
            pl.BlockSpec((1, BN), lane),
            pl.BlockSpec((1, BN), lane),
            pl.BlockSpec((1, BN), lane),
            pl.BlockSpec((1, BN), lane),
            pl.BlockSpec(memory_space=pltpu.SMEM),
            pl.BlockSpec(memory_space=pltpu.SMEM),
            pl.BlockSpec(memory_space=pltpu.SMEM),
            pl.BlockSpec(memory_space=pltpu.SMEM),
        ],
        out_shape=[
            jax.ShapeDtypeStruct((NP, D), jnp.float32),
            jax.ShapeDtypeStruct((NP, D), jnp.float32),
            jax.ShapeDtypeStruct((1, NP), jnp.float32),
            jax.ShapeDtypeStruct((1, NP), jnp.float32),
            jax.ShapeDtypeStruct((1, NP), jnp.float32),
            jax.ShapeDtypeStruct((1, NP), jnp.float32),
            jax.ShapeDtypeStruct((1, 1), jnp.float32),
            jax.ShapeDtypeStruct((1, 1), jnp.float32),
            jax.ShapeDtypeStruct((1, 1), jnp.float32),
            jax.ShapeDtypeStruct((1, 1), jnp.float32),
        ],
    )(oi, si, op, sp, b1i, b1p, fha, w2i, w2p, asi, adi, asp, adp)


def _head_body(oi_ref, si_ref, op_ref, sp_ref, b2i_ref, b2p_ref, aw_ref,
               mw_ref, mb_ref, y_ref):
    xi = oi_ref[...] / (si_ref[...] + EPS) + b2i_ref[...]
    xp = op_ref[...] / (sp_ref[...] + EPS) + b2p_ref[...]
    aw = aw_ref[...]
    wi = jnp.sum(xi * aw[0:1, :], axis=1, keepdims=True)
    wp = jnp.sum(xp * aw[1:2, :], axis=1, keepdims=True)
    mx = jnp.maximum(wi, wp)
    e0 = jnp.exp(wi - mx)
    e1 = jnp.exp(wp - mx)
    x = (e0 * xi + e1 * xp) / (e0 + e1)
    y_ref[...] = jnp.dot(x, mw_ref[...], preferred_element_type=jnp.float32) + mb_ref[...]


def _head(oi, si, op, sp, b2i, b2p, aw, mw, mb):
    grid = (NP // BN,)
    row = lambda i: (i, 0)
    full = lambda i: (0, 0)
    return pl.pallas_call(
        _head_body,
        grid=grid,
        in_specs=[
            pl.BlockSpec((BN, D), row),
            pl.BlockSpec((BN, 1), row),
            pl.BlockSpec((BN, D), row),
            pl.BlockSpec((BN, 1), row),
            pl.BlockSpec((1, D), full),
            pl.BlockSpec((1, D), full),
            pl.BlockSpec((2, D), full),
            pl.BlockSpec((D, 1), full),
            pl.BlockSpec((1, 1), full),
        ],
        out_specs=pl.BlockSpec((BN, 1), row),
        out_shape=jax.ShapeDtypeStruct((NP, 1), jnp.float32),
    )(oi, si, op, sp, b2i, b2p, aw, mw, mb)


def kernel(x_industry, edge_index_industry, x_pos_corr, edge_index_pos,
           W1i, a1i_s, a1i_d, b1i, W2i, a2i_s, a2i_d, b2i,
           W1p, a1p_s, a1p_d, b1p, W2p, a2p_s, a2p_d, b2p,
           fha, aw, mlp_W, mlp_b):
    f32 = jnp.float32
    loop = jnp.arange(N, dtype=jnp.int32)
    zpad = jnp.zeros((EEP - EE,), jnp.int32)

    def edges(ei, off):
        src = jnp.concatenate([ei[0].astype(jnp.int32), loop, zpad]) + off
        dst = jnp.concatenate([ei[1].astype(jnp.int32), loop, zpad])
        return src, dst

    src_i, dst_i = edges(edge_index_industry, 0)
    src_p, dst_p = edges(edge_index_pos, NP)
    srcoff = jnp.concatenate([src_i, src_p])
    dst2 = jnp.concatenate([dst_i, dst_p])

    xpad = jnp.zeros((NP - N, D), f32)
    xi_in = jnp.concatenate([x_industry, xpad], axis=0)
    xp_in = jnp.concatenate([x_pos_corr, xpad], axis=0)

    r1 = lambda v: v.reshape(1, D)

    h1i, hs1i, hd1i, gs1i, gd1i = _dense1(xi_in, W1i, r1(a1i_s), r1(a1i_d))
    h1p, hs1p, hd1p, gs1p, gd1p = _dense1(xp_in, W1p, r1(a1p_s), r1(a1p_d))

    def gbound(gs, gd):
        t = gs[0, 0] + gd[0, 0]
        return jnp.broadcast_to(jnp.where(t >= 0.0, t, 0.2 * t), (16,))

    g1 = jnp.concatenate([gbound(gs1i, gd1i), gbound(gs1p, gd1p)])
    hscat1 = jnp.concatenate([hs1i[0], hs1p[0]])
    hdcat1 = jnp.concatenate([hd1i[0], hd1p[0]])
    hcat1 = jnp.concatenate([h1i, h1p], axis=0)

    out1, ssum1 = _aggregate(srcoff, dst2, hscat1, hdcat1, g1, hcat1)

    c1 = lambda v: v.reshape(NP, 1)
    h2i, h2p, hs2i, hd2i, hs2p, hd2p, gs2i, gd2i, gs2p, gd2p = _fuse2(
        out1[0], c1(ssum1[:NP]), out1[1], c1(ssum1[NP:]),
        r1(b1i), r1(b1p), fha, W2i, W2p,
        r1(a2i_s), r1(a2i_d), r1(a2p_s), r1(a2p_d))

    g2 = jnp.concatenate([gbound(gs2i, gd2i), gbound(gs2p, gd2p)])
    hscat2 = jnp.concatenate([hs2i[0], hs2p[0]])
    hdcat2 = jnp.concatenate([hd2i[0], hd2p[0]])
    hcat2 = jnp.concatenate([h2i, h2p], axis=0)

    out2, ssum2 = _aggregate(srcoff, dst2, hscat2, hdcat2, g2, hcat2)

    y = _head(out2[0], c1(ssum2[:NP]), out2[1], c1(ssum2[NP:]),
              r1(b2i), r1(b2p), aw, mlp_W, mlp_b.reshape(1, 1))
    return y[:N]


# speedup vs baseline: 35.5111x; 1.1634x over previous
import functools

import jax
import jax.numpy as jnp
from jax import lax
from jax.experimental import pallas as pl
from jax.experimental.pallas import tpu as pltpu
from jax.experimental.pallas import tpu_sc as plsc

N = 10000
NP = 10240
D = 128
EE = 330000
NTILES = 16
T = 20736
B = 96
NCHUNK = T // B
EEP = NTILES * T
NTS = NP // NTILES
BN = 1024
EPS = 1e-16


def _dense_body(x_ref, w_ref, as_ref, ad_ref, h_ref, hs_ref, hd_ref, gs_ref, gd_ref):
    i = pl.program_id(0)
    h = jnp.dot(x_ref[...], w_ref[...], preferred_element_type=jnp.float32)
    h_ref[...] = h
    hs = jnp.sum(h * as_ref[...], axis=1)
    hd = jnp.sum(h * ad_ref[...], axis=1)
    hs_ref[...] = hs.reshape(1, BN)
    hd_ref[...] = hd.reshape(1, BN)
    bs = jnp.max(hs)
    bd = jnp.max(hd)

    @pl.when(i == 0)
    def _():
        gs_ref[0, 0] = bs
        gd_ref[0, 0] = bd

    @pl.when(i > 0)
    def _():
        gs_ref[0, 0] = jnp.maximum(gs_ref[0, 0], bs)
        gd_ref[0, 0] = jnp.maximum(gd_ref[0, 0], bd)


def _dense1(x, w, a_s, a_d):
    grid = (NP // BN,)
    return pl.pallas_call(
        _dense_body,
        grid=grid,
        in_specs=[
            pl.BlockSpec((BN, D), lambda i: (i, 0)),
            pl.BlockSpec((D, D), lambda i: (0, 0)),
            pl.BlockSpec((1, D), lambda i: (0, 0)),
            pl.BlockSpec((1, D), lambda i: (0, 0)),
        ],
        out_specs=[
            pl.BlockSpec((BN, D), lambda i: (i, 0)),
            pl.BlockSpec((1, BN), lambda i: (0, i)),
            pl.BlockSpec((1, BN), lambda i: (0, i)),
            pl.BlockSpec(memory_space=pltpu.SMEM),
            pl.BlockSpec(memory_space=pltpu.SMEM),
        ],
        out_shape=[
            jax.ShapeDtypeStruct((NP, D), jnp.float32),
            jax.ShapeDtypeStruct((1, NP), jnp.float32),
            jax.ShapeDtypeStruct((1, NP), jnp.float32),
            jax.ShapeDtypeStruct((1, 1), jnp.float32),
            jax.ShapeDtypeStruct((1, 1), jnp.float32),
        ],
    )(x, w, a_s, a_d)


def _agg_body(srcoff_hbm, dst_hbm, hs_hbm, hd_hbm, g_hbm, hcat_hbm,
              out_hbm, ssum_hbm,
              g_v, zs_v, bufs, out_sh, s_sh, hs_sh, hd_sh):
    c = lax.axis_index("c")
    s = lax.axis_index("s")

    pltpu.sync_copy(g_hbm.at[pl.ds(c * 16, 16)], g_v)
    g16 = g_v[...]
    coff = (c * NP).astype(jnp.int32)

    @pl.when(s == 0)
    def _():
        pltpu.sync_copy(hs_hbm, hs_sh)
        pltpu.sync_copy(hd_hbm, hd_sh)

    z16 = jnp.zeros((16,), jnp.float32)
    rows0 = bufs[0][6]

    def _zrow(r, carry):
        for k in range(D // 16):
            rows0[r, pl.ds(k * 16, 16)] = z16
        return carry

    lax.fori_loop(0, B, _zrow, 0)
    for k in range(NTS // 16):
        zs_v[pl.ds(k * 16, 16)] = z16
    for j in range(NTS // B):
        pltpu.sync_copy(rows0, out_sh.at[pl.ds(s * NTS + j * B, B)])
    rem = NTS - (NTS // B) * B
    if rem:
        pltpu.sync_copy(rows0.at[pl.ds(0, rem)],
                        out_sh.at[pl.ds(s * NTS + (NTS // B) * B, rem)])
    pltpu.sync_copy(zs_v, s_sh.at[pl.ds(s * NTS, NTS)])
    plsc.subcore_barrier()

    def _prefetch(ci, buf):
        src_b, dst_b, doff_b, hsg_b, hdg_b, ex_b, rows_b, semg, semh, semd, sems = buf
        base = s * T + ci * B
        pltpu.sync_copy(srcoff_hbm.at[pl.ds(c * EEP + base, B)], src_b)
        pltpu.sync_copy(dst_hbm.at[pl.ds(c * EEP + base, B)], dst_b)
        for j in range(B // 16):
            doff_b[pl.ds(j * 16, 16)] = dst_b[pl.ds(j * 16, 16)] + coff
        pltpu.async_copy(hcat_hbm.at[src_b], rows_b, semg)
        pltpu.async_copy(hs_sh.at[src_b], hsg_b, semh)
        pltpu.async_copy(hd_sh.at[doff_b], hdg_b, semd)

    def _wait_scatter(buf):
        src_b, dst_b, doff_b, hsg_b, hdg_b, ex_b, rows_b, semg, semh, semd, sems = buf
        pltpu.make_async_copy(rows_b, out_sh.at[dst_b], sems).wait()

    def _process(ci, buf):
        src_b, dst_b, doff_b, hsg_b, hdg_b, ex_b, rows_b, semg, semh, semd, sems = buf
        base = s * T + ci * B
        pltpu.make_async_copy(hs_sh.at[src_b], hsg_b, semh).wait()
        pltpu.make_async_copy(hd_sh.at[doff_b], hdg_b, semd).wait()
        for j in range(B // 16):
            t = hsg_b[pl.ds(j * 16, 16)] + hdg_b[pl.ds(j * 16, 16)]
            e = jnp.where(t >= 0.0, t, 0.2 * t)
            eid = base + j * 16 + lax.iota(jnp.int32, 16)
            e = jnp.where(eid < EE, e, -1e30)
            ex_b[pl.ds(j * 16, 16)] = jnp.exp(e - g16)
        pltpu.sync_copy(ex_b, s_sh.at[dst_b], add=True)
        pltpu.make_async_copy(hcat_hbm.at[src_b], rows_b, semg).wait()

        def _scale(r, inner):
            for u in range(2):
                exr = plsc.load_gather(
                    ex_b, (jnp.broadcast_to(2 * r + u, (16,)).astype(jnp.int32),))
                for k in range(D // 16):
                    rows_b[2 * r + u, pl.ds(k * 16, 16)] = (
                        rows_b[2 * r + u, pl.ds(k * 16, 16)] * exr)
            return inner

        lax.fori_loop(0, B // 2, _scale, 0)
        pltpu.async_copy(rows_b, out_sh.at[dst_b], sems, add=True)

    _prefetch(0, bufs[0])
    _prefetch(1, bufs[1])
    _process(0, bufs[0])
    _prefetch(2, bufs[2])
    _process(1, bufs[1])

    def _outer(gi, carry):
        for u in range(3):
            ci = 2 + 3 * gi + u
            bn = u
            _wait_scatter(bufs[bn])
            _prefetch(ci + 1, bufs[bn])
            _process(ci, bufs[(2 + u) % 3])
        return carry

    lax.fori_loop(0, (NCHUNK - 3) // 3, _outer, 0)
    _wait_scatter(bufs[NCHUNK % 3])
    _process(NCHUNK - 1, bufs[(NCHUNK - 1) % 3])
    _wait_scatter(bufs[(NCHUNK - 2) % 3])
    _wait_scatter(bufs[(NCHUNK - 1) % 3])
    plsc.subcore_barrier()

    pltpu.sync_copy(out_sh.at[pl.ds(s * NTS, NTS)],
                    out_hbm.at[c, pl.ds(s * NTS, NTS)])
    pltpu.sync_copy(s_sh.at[pl.ds(s * NTS, NTS)],
                    ssum_hbm.at[pl.ds(c * NP + s * NTS, NTS)])


def _buf_types():
    t = []
    for _ in range(3):
        t.append((
            pltpu.VMEM((B,), jnp.int32),
            pltpu.VMEM((B,), jnp.int32),
            pltpu.VMEM((B,), jnp.int32),
            pltpu.VMEM((B,), jnp.float32),
            pltpu.VMEM((B,), jnp.float32),
            pltpu.VMEM((B,), jnp.float32),
            pltpu.VMEM((B, D), jnp.float32),
            pltpu.SemaphoreType.DMA,
            pltpu.SemaphoreType.DMA,
            pltpu.SemaphoreType.DMA,
            pltpu.SemaphoreType.DMA,
        ))
    return tuple(t)


@functools.partial(
    pl.kernel,
    out_type=[
        jax.ShapeDtypeStruct((2, NP, D), jnp.float32),
        jax.ShapeDtypeStruct((2 * NP,), jnp.float32),
    ],
    mesh=plsc.VectorSubcoreMesh(core_axis_name="c", subcore_axis_name="s"),
    compiler_params=pltpu.CompilerParams(needs_layout_passes=False),
    scratch_types=[
        pltpu.VMEM((16,), jnp.float32),
        pltpu.VMEM((NTS,), jnp.float32),
        _buf_types(),
        pltpu.VMEM_SHARED((NP, D), jnp.float32),
        pltpu.VMEM_SHARED((NP,), jnp.float32),
        pltpu.VMEM_SHARED((2 * NP,), jnp.float32),
        pltpu.VMEM_SHARED((2 * NP,), jnp.float32),
    ],
)
def _aggregate(srcoff, dst, hscat, hdcat, g, hcat, out, ssum, *scratch):
    _agg_body(srcoff, dst, hscat, hdcat, g, hcat, out, ssum, *scratch)


def _fuse2_body(oi_ref, si_ref, op_ref, sp_ref, b1i_ref, b1p_ref, fha_ref,
                w2i_ref, w2p_ref, asi_ref, adi_ref, asp_ref, adp_ref,
                h2i_ref, h2p_ref, hs2i_ref, hd2i_ref, hs2p_ref, hd2p_ref,
                gsi_ref, gdi_ref, gsp_ref, gdp_ref):
    i = pl.program_id(0)
    xi = jax.nn.relu(oi_ref[...] / (si_ref[...] + EPS) + b1i_ref[...])
    xp = jax.nn.relu(op_ref[...] / (sp_ref[...] + EPS) + b1p_ref[...])
    fha = fha_ref[...]
    wi = jnp.sum(xi * fha[0:1, :], axis=1, keepdims=True)
    wp = jnp.sum(xp * fha[1:2, :], axis=1, keepdims=True)
    mx = jnp.maximum(wi, wp)
    e0 = jnp.exp(wi - mx)
    e1 = jnp.exp(wp - mx)
    xf = (e0 * xi + e1 * xp) / (e0 + e1)
    h2i = jnp.dot(xf, w2i_ref[...], preferred_element_type=jnp.float32)
    h2p = jnp.dot(xf, w2p_ref[...], preferred_element_type=jnp.float32)
    h2i_ref[...] = h2i
    h2p_ref[...] = h2p
    hs2i = jnp.sum(h2i * asi_ref[...], axis=1)
    hd2i = jnp.sum(h2i * adi_ref[...], axis=1)
    hs2p = jnp.sum(h2p * asp_ref[...], axis=1)
    hd2p = jnp.sum(h2p * adp_ref[...], axis=1)
    hs2i_ref[...] = hs2i.reshape(1, BN)
    hd2i_ref[...] = hd2i.reshape(1, BN)
    hs2p_ref[...] = hs2p.reshape(1, BN)
    hd2p_ref[...] = hd2p.reshape(1, BN)

    @pl.when(i == 0)
    def _():
        gsi_ref[0, 0] = jnp.max(hs2i)
        gdi_ref[0, 0] = jnp.max(hd2i)
        gsp_ref[0, 0] = jnp.max(hs2p)
        gdp_ref[0, 0] = jnp.max(hd2p)

    @pl.when(i > 0)
    def _():
        gsi_ref[0, 0] = jnp.maximum(gsi_ref[0, 0], jnp.max(hs2i))
        gdi_ref[0, 0] = jnp.maximum(gdi_ref[0, 0], jnp.max(hd2i))
        gsp_ref[0, 0] = jnp.maximum(gsp_ref[0, 0], jnp.max(hs2p))
        gdp_ref[0, 0] = jnp.maximum(gdp_ref[0, 0], jnp.max(hd2p))


def _fuse2(oi, si, op, sp, b1i, b1p, fha, w2i, w2p, asi, adi, asp, adp):
    grid = (NP // BN,)
    row = lambda i: (i, 0)
    full = lambda i: (0, 0)
    lane = lambda i: (0, i)
    return pl.pallas_call(
        _fuse2_body,
        grid=grid,
        in_specs=[
            pl.BlockSpec((BN, D), row),
            pl.BlockSpec((BN, 1), row),
            pl.BlockSpec((BN, D), row),
            pl.BlockSpec((BN, 1), row),
            pl.BlockSpec((1, D), full),
            pl.BlockSpec((1, D), full),
            pl.BlockSpec((2, D), full),
            pl.BlockSpec((D, D), full),
            pl.BlockSpec((D, D), full),
            pl.BlockSpec((1, D), full),
            pl.BlockSpec((1, D), full),
            pl.BlockSpec((1, D), full),
            pl.BlockSpec((1, D), full),
        ],
        out_specs=[
            pl.BlockSpec((BN, D), row),
            pl.BlockSpec((BN, D), row),
            pl.BlockSpec((1, BN), lane),
            pl.BlockSpec((1, BN), lane),
            pl.BlockSpec((1, BN), lane),
            pl.BlockSpec((1, BN), lane),
            pl.BlockSpec(memory_space=pltpu.SMEM),
            pl.BlockSpec(memory_space=pltpu.SMEM),
            pl.BlockSpec(memory_space=pltpu.SMEM),
            pl.BlockSpec(memory_space=pltpu.SMEM),
        ],
        out_shape=[
            jax.ShapeDtypeStruct((NP, D), jnp.float32),
            jax.ShapeDtypeStruct((NP, D), jnp.float32),
            jax.ShapeDtypeStruct((1, NP), jnp.float32),
            jax.ShapeDtypeStruct((1, NP), jnp.float32),
            jax.ShapeDtypeStruct((1, NP), jnp.float32),
            jax.ShapeDtypeStruct((1, NP), jnp.float32),
            jax.ShapeDtypeStruct((1, 1), jnp.float32),
            jax.ShapeDtypeStruct((1, 1), jnp.float32),
            jax.ShapeDtypeStruct((1, 1), jnp.float32),
            jax.ShapeDtypeStruct((1, 1), jnp.float32),
        ],
    )(oi, si, op, sp, b1i, b1p, fha, w2i, w2p, asi, adi, asp, adp)


def _head_body(oi_ref, si_ref, op_ref, sp_ref, b2i_ref, b2p_ref, aw_ref,
               mw_ref, mb_ref, y_ref):
    xi = oi_ref[...] / (si_ref[...] + EPS) + b2i_ref[...]
    xp = op_ref[...] / (sp_ref[...] + EPS) + b2p_ref[...]
    aw = aw_ref[...]
    wi = jnp.sum(xi * aw[0:1, :], axis=1, keepdims=True)
    wp = jnp.sum(xp * aw[1:2, :], axis=1, keepdims=True)
    mx = jnp.maximum(wi, wp)
    e0 = jnp.exp(wi - mx)
    e1 = jnp.exp(wp - mx)
    x = (e0 * xi + e1 * xp) / (e0 + e1)
    y_ref[...] = jnp.dot(x, mw_ref[...], preferred_element_type=jnp.float32) + mb_ref[...]


def _head(oi, si, op, sp, b2i, b2p, aw, mw, mb):
    grid = (NP // BN,)
    row = lambda i: (i, 0)
    full = lambda i: (0, 0)
    return pl.pallas_call(
        _head_body,
        grid=grid,
        in_specs=[
            pl.BlockSpec((BN, D), row),
            pl.BlockSpec((BN, 1), row),
            pl.BlockSpec((BN, D), row),
            pl.BlockSpec((BN, 1), row),
            pl.BlockSpec((1, D), full),
            pl.BlockSpec((1, D), full),
            pl.BlockSpec((2, D), full),
            pl.BlockSpec((D, 1), full),
            pl.BlockSpec((1, 1), full),
        ],
        out_specs=pl.BlockSpec((BN, 1), row),
        out_shape=jax.ShapeDtypeStruct((NP, 1), jnp.float32),
    )(oi, si, op, sp, b2i, b2p, aw, mw, mb)


def kernel(x_industry, edge_index_industry, x_pos_corr, edge_index_pos,
           W1i, a1i_s, a1i_d, b1i, W2i, a2i_s, a2i_d, b2i,
           W1p, a1p_s, a1p_d, b1p, W2p, a2p_s, a2p_d, b2p,
           fha, aw, mlp_W, mlp_b):
    f32 = jnp.float32
    loop = jnp.arange(N, dtype=jnp.int32)
    zpad = jnp.zeros((EEP - EE,), jnp.int32)

    def edges(ei, off):
        src = jnp.concatenate([ei[0].astype(jnp.int32), loop, zpad]) + off
        dst = jnp.concatenate([ei[1].astype(jnp.int32), loop, zpad])
        return src, dst

    src_i, dst_i = edges(edge_index_industry, 0)
    src_p, dst_p = edges(edge_index_pos, NP)
    srcoff = jnp.concatenate([src_i, src_p])
    dst2 = jnp.concatenate([dst_i, dst_p])

    xpad = jnp.zeros((NP - N, D), f32)
    xi_in = jnp.concatenate([x_industry, xpad], axis=0)
    xp_in = jnp.concatenate([x_pos_corr, xpad], axis=0)

    r1 = lambda v: v.reshape(1, D)

    h1i, hs1i, hd1i, gs1i, gd1i = _dense1(xi_in, W1i, r1(a1i_s), r1(a1i_d))
    h1p, hs1p, hd1p, gs1p, gd1p = _dense1(xp_in, W1p, r1(a1p_s), r1(a1p_d))

    def gbound(gs, gd):
        t = gs[0, 0] + gd[0, 0]
        return jnp.broadcast_to(jnp.where(t >= 0.0, t, 0.2 * t), (16,))

    g1 = jnp.concatenate([gbound(gs1i, gd1i), gbound(gs1p, gd1p)])
    hscat1 = jnp.concatenate([hs1i[0], hs1p[0]])
    hdcat1 = jnp.concatenate([hd1i[0], hd1p[0]])
    hcat1 = jnp.concatenate([h1i, h1p], axis=0)

    out1, ssum1 = _aggregate(srcoff, dst2, hscat1, hdcat1, g1, hcat1)

    c1 = lambda v: v.reshape(NP, 1)
    h2i, h2p, hs2i, hd2i, hs2p, hd2p, gs2i, gd2i, gs2p, gd2p = _fuse2(
        out1[0], c1(ssum1[:NP]), out1[1], c1(ssum1[NP:]),
        r1(b1i), r1(b1p), fha, W2i, W2p,
        r1(a2i_s), r1(a2i_d), r1(a2p_s), r1(a2p_d))

    g2 = jnp.concatenate([gbound(gs2i, gd2i), gbound(gs2p, gd2p)])
    hscat2 = jnp.concatenate([hs2i[0], hs2p[0]])
    hdcat2 = jnp.concatenate([hd2i[0], hd2p[0]])
    hcat2 = jnp.concatenate([h2i, h2p], axis=0)

    out2, ssum2 = _aggregate(srcoff, dst2, hscat2, hdcat2, g2, hcat2)

    y = _head(out2[0], c1(ssum2[:NP]), out2[1], c1(ssum2[NP:]),
              r1(b2i), r1(b2p), aw, mlp_W, mlp_b.reshape(1, 1))
    return y[:N]


# speedup vs baseline: 37.0259x; 1.0427x over previous
import functools

import jax
import jax.numpy as jnp
from jax import lax
from jax.experimental import pallas as pl
from jax.experimental.pallas import tpu as pltpu
from jax.experimental.pallas import tpu_sc as plsc

N = 10000
NP = 10240
D = 128
EE = 330000
NTILES = 16
T = 20736
B = 96
NCHUNK = T // B
EEP = NTILES * T
NTS = NP // NTILES
BN = 1024
EPS = 1e-16


def _dense_body(x_ref, w_ref, as_ref, ad_ref, h_ref, hs_ref, hd_ref, gs_ref, gd_ref):
    i = pl.program_id(0)
    h = jnp.dot(x_ref[...], w_ref[...], preferred_element_type=jnp.float32)
    h_ref[...] = h
    hs = jnp.sum(h * as_ref[...], axis=1)
    hd = jnp.sum(h * ad_ref[...], axis=1)
    hs_ref[...] = hs.reshape(1, BN)
    hd_ref[...] = hd.reshape(1, BN)
    bs = jnp.max(hs)
    bd = jnp.max(hd)

    @pl.when(i == 0)
    def _():
        gs_ref[0, 0] = bs
        gd_ref[0, 0] = bd

    @pl.when(i > 0)
    def _():
        gs_ref[0, 0] = jnp.maximum(gs_ref[0, 0], bs)
        gd_ref[0, 0] = jnp.maximum(gd_ref[0, 0], bd)


def _dense1(x, w, a_s, a_d):
    grid = (NP // BN,)
    return pl.pallas_call(
        _dense_body,
        grid=grid,
        in_specs=[
            pl.BlockSpec((BN, D), lambda i: (i, 0)),
            pl.BlockSpec((D, D), lambda i: (0, 0)),
            pl.BlockSpec((1, D), lambda i: (0, 0)),
            pl.BlockSpec((1, D), lambda i: (0, 0)),
        ],
        out_specs=[
            pl.BlockSpec((BN, D), lambda i: (i, 0)),
            pl.BlockSpec((1, BN), lambda i: (0, i)),
            pl.BlockSpec((1, BN), lambda i: (0, i)),
            pl.BlockSpec(memory_space=pltpu.SMEM),
            pl.BlockSpec(memory_space=pltpu.SMEM),
        ],
        out_shape=[
            jax.ShapeDtypeStruct((NP, D), jnp.float32),
            jax.ShapeDtypeStruct((1, NP), jnp.float32),
            jax.ShapeDtypeStruct((1, NP), jnp.float32),
            jax.ShapeDtypeStruct((1, 1), jnp.float32),
            jax.ShapeDtypeStruct((1, 1), jnp.float32),
        ],
    )(x, w, a_s, a_d)


def _agg_body(srcoff_hbm, dst_hbm, hs_hbm, hd_hbm, g_hbm, hcat_hbm,
              out_hbm, ssum_hbm,
              g_v, zs_v, bufs, out_sh, s_sh, hs_sh, hd_sh):
    c = lax.axis_index("c")
    s = lax.axis_index("s")

    pltpu.sync_copy(g_hbm.at[pl.ds(c * 16, 16)], g_v)
    g16 = g_v[...]
    coff = (c * NP).astype(jnp.int32)

    @pl.when(s == 0)
    def _():
        pltpu.sync_copy(hs_hbm, hs_sh)
        pltpu.sync_copy(hd_hbm, hd_sh)

    z16 = jnp.zeros((16,), jnp.float32)
    rows0 = bufs[0][6]

    def _zrow(r, carry):
        for k in range(D // 16):
            rows0[r, pl.ds(k * 16, 16)] = z16
        return carry

    lax.fori_loop(0, B, _zrow, 0)
    for k in range(NTS // 16):
        zs_v[pl.ds(k * 16, 16)] = z16
    for j in range(NTS // B):
        pltpu.sync_copy(rows0, out_sh.at[pl.ds(s * NTS + j * B, B)])
    rem = NTS - (NTS // B) * B
    if rem:
        pltpu.sync_copy(rows0.at[pl.ds(0, rem)],
                        out_sh.at[pl.ds(s * NTS + (NTS // B) * B, rem)])
    pltpu.sync_copy(zs_v, s_sh.at[pl.ds(s * NTS, NTS)])
    plsc.subcore_barrier()

    def _prefetch(ci, buf):
        src_b, dst_b, doff_b, hsg_b, hdg_b, ex_b, rows_b, semg, semh, semd, sems, semx = buf
        base = s * T + ci * B
        pltpu.sync_copy(srcoff_hbm.at[pl.ds(c * EEP + base, B)], src_b)
        pltpu.sync_copy(dst_hbm.at[pl.ds(c * EEP + base, B)], dst_b)
        for j in range(B // 16):
            doff_b[pl.ds(j * 16, 16)] = dst_b[pl.ds(j * 16, 16)] + coff
        pltpu.async_copy(hcat_hbm.at[src_b], rows_b, semg)
        pltpu.async_copy(hs_sh.at[src_b], hsg_b, semh)
        pltpu.async_copy(hd_sh.at[doff_b], hdg_b, semd)

    def _wait_scatter(buf):
        src_b, dst_b, doff_b, hsg_b, hdg_b, ex_b, rows_b, semg, semh, semd, sems, semx = buf
        pltpu.make_async_copy(ex_b, s_sh.at[dst_b], semx).wait()
        pltpu.make_async_copy(rows_b, out_sh.at[dst_b], sems).wait()

    def _process(ci, buf):
        src_b, dst_b, doff_b, hsg_b, hdg_b, ex_b, rows_b, semg, semh, semd, sems, semx = buf
        base = s * T + ci * B
        pltpu.make_async_copy(hs_sh.at[src_b], hsg_b, semh).wait()
        pltpu.make_async_copy(hd_sh.at[doff_b], hdg_b, semd).wait()
        for j in range(B // 16):
            t = hsg_b[pl.ds(j * 16, 16)] + hdg_b[pl.ds(j * 16, 16)]
            e = jnp.where(t >= 0.0, t, 0.2 * t)
            eid = base + j * 16 + lax.iota(jnp.int32, 16)
            e = jnp.where(eid < EE, e, -1e30)
            ex_b[pl.ds(j * 16, 16)] = jnp.exp(e - g16)
        pltpu.async_copy(ex_b, s_sh.at[dst_b], semx, add=True)
        pltpu.make_async_copy(hcat_hbm.at[src_b], rows_b, semg).wait()

        def _scale(r, inner):
            for u in range(4):
                exr = plsc.load_gather(
                    ex_b, (jnp.broadcast_to(4 * r + u, (16,)).astype(jnp.int32),))
                for k in range(D // 16):
                    rows_b[4 * r + u, pl.ds(k * 16, 16)] = (
                        rows_b[4 * r + u, pl.ds(k * 16, 16)] * exr)
            return inner

        lax.fori_loop(0, B // 4, _scale, 0)
        pltpu.async_copy(rows_b, out_sh.at[dst_b], sems, add=True)

    _prefetch(0, bufs[0])
    _prefetch(1, bufs[1])
    _process(0, bufs[0])
    _prefetch(2, bufs[2])
    _process(1, bufs[1])

    def _outer(gi, carry):
        for u in range(3):
            ci = 2 + 3 * gi + u
            bn = u
            _wait_scatter(bufs[bn])
            _prefetch(ci + 1, bufs[bn])
            _process(ci, bufs[(2 + u) % 3])
        return carry

    lax.fori_loop(0, (NCHUNK - 3) // 3, _outer, 0)
    _wait_scatter(bufs[NCHUNK % 3])
    _process(NCHUNK - 1, bufs[(NCHUNK - 1) % 3])
    _wait_scatter(bufs[(NCHUNK - 2) % 3])
    _wait_scatter(bufs[(NCHUNK - 1) % 3])
    plsc.subcore_barrier()

    pltpu.sync_copy(out_sh.at[pl.ds(s * NTS, NTS)],
                    out_hbm.at[c, pl.ds(s * NTS, NTS)])
    pltpu.sync_copy(s_sh.at[pl.ds(s * NTS, NTS)],
                    ssum_hbm.at[pl.ds(c * NP + s * NTS, NTS)])


def _buf_types():
    t = []
    for _ in range(3):
        t.append((
            pltpu.VMEM((B,), jnp.int32),
            pltpu.VMEM((B,), jnp.int32),
            pltpu.VMEM((B,), jnp.int32),
            pltpu.VMEM((B,), jnp.float32),
            pltpu.VMEM((B,), jnp.float32),
            pltpu.VMEM((B,), jnp.float32),
            pltpu.VMEM((B, D), jnp.float32),
            pltpu.SemaphoreType.DMA,
            pltpu.SemaphoreType.DMA,
            pltpu.SemaphoreType.DMA,
            pltpu.SemaphoreType.DMA,
            pltpu.SemaphoreType.DMA,
        ))
    return tuple(t)


@functools.partial(
    pl.kernel,
    out_type=[
        jax.ShapeDtypeStruct((2, NP, D), jnp.float32),
        jax.ShapeDtypeStruct((2 * NP,), jnp.float32),
    ],
    mesh=plsc.VectorSubcoreMesh(core_axis_name="c", subcore_axis_name="s"),
    compiler_params=pltpu.CompilerParams(needs_layout_passes=False),
    scratch_types=[
        pltpu.VMEM((16,), jnp.float32),
        pltpu.VMEM((NTS,), jnp.float32),
        _buf_types(),
        pltpu.VMEM_SHARED((NP, D), jnp.float32),
        pltpu.VMEM_SHARED((NP,), jnp.float32),
        pltpu.VMEM_SHARED((2 * NP,), jnp.float32),
        pltpu.VMEM_SHARED((2 * NP,), jnp.float32),
    ],
)
def _aggregate(srcoff, dst, hscat, hdcat, g, hcat, out, ssum, *scratch):
    _agg_body(srcoff, dst, hscat, hdcat, g, hcat, out, ssum, *scratch)


def _fuse2_body(oi_ref, si_ref, op_ref, sp_ref, b1i_ref, b1p_ref, fha_ref,
                w2i_ref, w2p_ref, asi_ref, adi_ref, asp_ref, adp_ref,
                h2i_ref, h2p_ref, hs2i_ref, hd2i_ref, hs2p_ref, hd2p_ref,
                gsi_ref, gdi_ref, gsp_ref, gdp_ref):
    i = pl.program_id(0)
    xi = jax.nn.relu(oi_ref[...] / (si_ref[...] + EPS) + b1i_ref[...])
    xp = jax.nn.relu(op_ref[...] / (sp_ref[...] + EPS) + b1p_ref[...])
    fha = fha_ref[...]
    wi = jnp.sum(xi * fha[0:1, :], axis=1, keepdims=True)
    wp = jnp.sum(xp * fha[1:2, :], axis=1, keepdims=True)
    mx = jnp.maximum(wi, wp)
    e0 = jnp.exp(wi - mx)
    e1 = jnp.exp(wp - mx)
    xf = (e0 * xi + e1 * xp) / (e0 + e1)
    h2i = jnp.dot(xf, w2i_ref[...], preferred_element_type=jnp.float32)
    h2p = jnp.dot(xf, w2p_ref[...], preferred_element_type=jnp.float32)
    h2i_ref[...] = h2i
    h2p_ref[...] = h2p
    hs2i = jnp.sum(h2i * asi_ref[...], axis=1)
    hd2i = jnp.sum(h2i * adi_ref[...], axis=1)
    hs2p = jnp.sum(h2p * asp_ref[...], axis=1)
    hd2p = jnp.sum(h2p * adp_ref[...], axis=1)
    hs2i_ref[...] = hs2i.reshape(1, BN)
    hd2i_ref[...] = hd2i.reshape(1, BN)
    hs2p_ref[...] = hs2p.reshape(1, BN)
    hd2p_ref[...] = hd2p.reshape(1, BN)

    @pl.when(i == 0)
    def _():
        gsi_ref[0, 0] = jnp.max(hs2i)
        gdi_ref[0, 0] = jnp.max(hd2i)
        gsp_ref[0, 0] = jnp.max(hs2p)
        gdp_ref[0, 0] = jnp.max(hd2p)

    @pl.when(i > 0)
    def _():
        gsi_ref[0, 0] = jnp.maximum(gsi_ref[0, 0], jnp.max(hs2i))
        gdi_ref[0, 0] = jnp.maximum(gdi_ref[0, 0], jnp.max(hd2i))
        gsp_ref[0, 0] = jnp.maximum(gsp_ref[0, 0], jnp.max(hs2p))
        gdp_ref[0, 0] = jnp.maximum(gdp_ref[0, 0], jnp.max(hd2p))


def _fuse2(oi, si, op, sp, b1i, b1p, fha, w2i, w2p, asi, adi, asp, adp):
    grid = (NP // BN,)
    row = lambda i: (i, 0)
    full = lambda i: (0, 0)
    lane = lambda i: (0, i)
    return pl.pallas_call(
        _fuse2_body,
        grid=grid,
        in_specs=[
            pl.BlockSpec((BN, D), row),
            pl.BlockSpec((BN, 1), row),
            pl.BlockSpec((BN, D), row),
            pl.BlockSpec((BN, 1), row),
            pl.BlockSpec((1, D), full),
            pl.BlockSpec((1, D), full),
            pl.BlockSpec((2, D), full),
            pl.BlockSpec((D, D), full),
            pl.BlockSpec((D, D), full),
            pl.BlockSpec((1, D), full),
            pl.BlockSpec((1, D), full),
            pl.BlockSpec((1, D), full),
            pl.BlockSpec((1, D), full),
        ],
        out_specs=[
            pl.BlockSpec((BN, D), row),
            pl.BlockSpec((BN, D), row),
            pl.BlockSpec((1, BN), lane),
            pl.BlockSpec((1, BN), lane),
            pl.BlockSpec((1, BN), lane),
            pl.BlockSpec((1, BN), lane),
            pl.BlockSpec(memory_space=pltpu.SMEM),
            pl.BlockSpec(memory_space=pltpu.SMEM),
            pl.BlockSpec(memory_space=pltpu.SMEM),
            pl.BlockSpec(memory_space=pltpu.SMEM),
        ],
        out_shape=[
            jax.ShapeDtypeStruct((NP, D), jnp.float32),
            jax.ShapeDtypeStruct((NP, D), jnp.float32),
            jax.ShapeDtypeStruct((1, NP), jnp.float32),
            jax.ShapeDtypeStruct((1, NP), jnp.float32),
            jax.ShapeDtypeStruct((1, NP), jnp.float32),
            jax.ShapeDtypeStruct((1, NP), jnp.float32),
            jax.ShapeDtypeStruct((1, 1), jnp.float32),
            jax.ShapeDtypeStruct((1, 1), jnp.float32),
            jax.ShapeDtypeStruct((1, 1), jnp.float32),
            jax.ShapeDtypeStruct((1, 1), jnp.float32),
        ],
    )(oi, si, op, sp, b1i, b1p, fha, w2i, w2p, asi, adi, asp, adp)


def _head_body(oi_ref, si_ref, op_ref, sp_ref, b2i_ref, b2p_ref, aw_ref,
               mw_ref, mb_ref, y_ref):
    xi = oi_ref[...] / (si_ref[...] + EPS) + b2i_ref[...]
    xp = op_ref[...] / (sp_ref[...] + EPS) + b2p_ref[...]
    aw = aw_ref[...]
    wi = jnp.sum(xi * aw[0:1, :], axis=1, keepdims=True)
    wp = jnp.sum(xp * aw[1:2, :], axis=1, keepdims=True)
    mx = jnp.maximum(wi, wp)
    e0 = jnp.exp(wi - mx)
    e1 = jnp.exp(wp - mx)
    x = (e0 * xi + e1 * xp) / (e0 + e1)
    y_ref[...] = jnp.dot(x, mw_ref[...], preferred_element_type=jnp.float32) + mb_ref[...]


def _head(oi, si, op, sp, b2i, b2p, aw, mw, mb):
    grid = (NP // BN,)
    row = lambda i: (i, 0)
    full = lambda i: (0, 0)
    return pl.pallas_call(
        _head_body,
        grid=grid,
        in_specs=[
            pl.BlockSpec((BN, D), row),
            pl.BlockSpec((BN, 1), row),
            pl.BlockSpec((BN, D), row),
            pl.BlockSpec((BN, 1), row),
            pl.BlockSpec((1, D), full),
            pl.BlockSpec((1, D), full),
            pl.BlockSpec((2, D), full),
            pl.BlockSpec((D, 1), full),
            pl.BlockSpec((1, 1), full),
        ],
        out_specs=pl.BlockSpec((BN, 1), row),
        out_shape=jax.ShapeDtypeStruct((NP, 1), jnp.float32),
    )(oi, si, op, sp, b2i, b2p, aw, mw, mb)


def kernel(x_industry, edge_index_industry, x_pos_corr, edge_index_pos,
           W1i, a1i_s, a1i_d, b1i, W2i, a2i_s, a2i_d, b2i,
           W1p, a1p_s, a1p_d, b1p, W2p, a2p_s, a2p_d, b2p,
           fha, aw, mlp_W, mlp_b):
    f32 = jnp.float32
    loop = jnp.arange(N, dtype=jnp.int32)
    zpad = jnp.zeros((EEP - EE,), jnp.int32)

    def edges(ei, off):
        src = jnp.concatenate([ei[0].astype(jnp.int32), loop, zpad]) + off
        dst = jnp.concatenate([ei[1].astype(jnp.int32), loop, zpad])
        return src, dst

    src_i, dst_i = edges(edge_index_industry, 0)
    src_p, dst_p = edges(edge_index_pos, NP)
    srcoff = jnp.concatenate([src_i, src_p])
    dst2 = jnp.concatenate([dst_i, dst_p])

    xpad = jnp.zeros((NP - N, D), f32)
    xi_in = jnp.concatenate([x_industry, xpad], axis=0)
    xp_in = jnp.concatenate([x_pos_corr, xpad], axis=0)

    r1 = lambda v: v.reshape(1, D)

    h1i, hs1i, hd1i, gs1i, gd1i = _dense1(xi_in, W1i, r1(a1i_s), r1(a1i_d))
    h1p, hs1p, hd1p, gs1p, gd1p = _dense1(xp_in, W1p, r1(a1p_s), r1(a1p_d))

    def gbound(gs, gd):
        t = gs[0, 0] + gd[0, 0]
        return jnp.broadcast_to(jnp.where(t >= 0.0, t, 0.2 * t), (16,))

    g1 = jnp.concatenate([gbound(gs1i, gd1i), gbound(gs1p, gd1p)])
    hscat1 = jnp.concatenate([hs1i[0], hs1p[0]])
    hdcat1 = jnp.concatenate([hd1i[0], hd1p[0]])
    hcat1 = jnp.concatenate([h1i, h1p], axis=0)

    out1, ssum1 = _aggregate(srcoff, dst2, hscat1, hdcat1, g1, hcat1)

    c1 = lambda v: v.reshape(NP, 1)
    h2i, h2p, hs2i, hd2i, hs2p, hd2p, gs2i, gd2i, gs2p, gd2p = _fuse2(
        out1[0], c1(ssum1[:NP]), out1[1], c1(ssum1[NP:]),
        r1(b1i), r1(b1p), fha, W2i, W2p,
        r1(a2i_s), r1(a2i_d), r1(a2p_s), r1(a2p_d))

    g2 = jnp.concatenate([gbound(gs2i, gd2i), gbound(gs2p, gd2p)])
    hscat2 = jnp.concatenate([hs2i[0], hs2p[0]])
    hdcat2 = jnp.concatenate([hd2i[0], hd2p[0]])
    hcat2 = jnp.concatenate([h2i, h2p], axis=0)

    out2, ssum2 = _aggregate(srcoff, dst2, hscat2, hdcat2, g2, hcat2)

    y = _head(out2[0], c1(ssum2[:NP]), out2[1], c1(ssum2[NP:]),
              r1(b2i), r1(b2p), aw, mlp_W, mlp_b.reshape(1, 1))
    return y[:N]


# speedup vs baseline: 41.5261x; 1.1215x over previous
import functools

import jax
import jax.numpy as jnp
from jax import lax
from jax.experimental import pallas as pl
from jax.experimental.pallas import tpu as pltpu
from jax.experimental.pallas import tpu_sc as plsc

N = 10000
NP = 10240
D = 128
EE = 330000
NTILES = 16
T = 20736
B = 96
NCHUNK = T // B
EEP = NTILES * T
NTS = NP // NTILES
BN = 1024
EPS = 1e-16


def _dense_body(x_ref, w_ref, as_ref, ad_ref, h_ref, hs_ref, hd_ref, gs_ref, gd_ref):
    i = pl.program_id(0)
    h = jnp.dot(x_ref[...], w_ref[...], preferred_element_type=jnp.float32)
    h_ref[...] = h
    hs = jnp.sum(h * as_ref[...], axis=1)
    hd = jnp.sum(h * ad_ref[...], axis=1)
    hs_ref[...] = hs.reshape(1, BN)
    hd_ref[...] = hd.reshape(1, BN)
    bs = jnp.max(hs)
    bd = jnp.max(hd)

    @pl.when(i == 0)
    def _():
        gs_ref[0, 0] = bs
        gd_ref[0, 0] = bd

    @pl.when(i > 0)
    def _():
        gs_ref[0, 0] = jnp.maximum(gs_ref[0, 0], bs)
        gd_ref[0, 0] = jnp.maximum(gd_ref[0, 0], bd)


def _dense1(x, w, a_s, a_d):
    grid = (NP // BN,)
    return pl.pallas_call(
        _dense_body,
        grid=grid,
        in_specs=[
            pl.BlockSpec((BN, D), lambda i: (i, 0)),
            pl.BlockSpec((D, D), lambda i: (0, 0)),
            pl.BlockSpec((1, D), lambda i: (0, 0)),
            pl.BlockSpec((1, D), lambda i: (0, 0)),
        ],
        out_specs=[
            pl.BlockSpec((BN, D), lambda i: (i, 0)),
            pl.BlockSpec((1, BN), lambda i: (0, i)),
            pl.BlockSpec((1, BN), lambda i: (0, i)),
            pl.BlockSpec(memory_space=pltpu.SMEM),
            pl.BlockSpec(memory_space=pltpu.SMEM),
        ],
        out_shape=[
            jax.ShapeDtypeStruct((NP, D), jnp.float32),
            jax.ShapeDtypeStruct((1, NP), jnp.float32),
            jax.ShapeDtypeStruct((1, NP), jnp.float32),
            jax.ShapeDtypeStruct((1, 1), jnp.float32),
            jax.ShapeDtypeStruct((1, 1), jnp.float32),
        ],
    )(x, w, a_s, a_d)


def _agg_body(eidx_hbm, hs_hbm, hd_hbm, g_hbm, hcat_hbm,
              out_hbm, ssum_hbm,
              g_v, zs_v, bufs, out_sh, s_sh, hs_sh, hd_sh):
    c = lax.axis_index("c")
    s = lax.axis_index("s")

    pltpu.sync_copy(g_hbm.at[pl.ds(c * 16, 16)], g_v)
    g16 = g_v[...]
    coff = (c * NP).astype(jnp.int32)

    @pl.when(s == 0)
    def _():
        pltpu.sync_copy(hs_hbm, hs_sh)
        pltpu.sync_copy(hd_hbm, hd_sh)

    z16 = jnp.zeros((16,), jnp.float32)
    rows0 = bufs[0][5]

    def _zrow(r, carry):
        for k in range(D // 16):
            rows0[r, pl.ds(k * 16, 16)] = z16
        return carry

    lax.fori_loop(0, B, _zrow, 0)
    for k in range(NTS // 16):
        zs_v[pl.ds(k * 16, 16)] = z16
    for j in range(NTS // B):
        pltpu.sync_copy(rows0, out_sh.at[pl.ds(s * NTS + j * B, B)])
    rem = NTS - (NTS // B) * B
    if rem:
        pltpu.sync_copy(rows0.at[pl.ds(0, rem)],
                        out_sh.at[pl.ds(s * NTS + (NTS // B) * B, rem)])
    pltpu.sync_copy(zs_v, s_sh.at[pl.ds(s * NTS, NTS)])
    plsc.subcore_barrier()

    def _prefetch(ci, buf):
        sd_b, doff_b, hsg_b, hdg_b, ex_b, rows_b, semg, semh, semd, sems, semx = buf
        rowi = (c * NTILES + s) * NCHUNK + ci
        pltpu.sync_copy(eidx_hbm.at[rowi], sd_b)
        for j in range(B // 16):
            doff_b[pl.ds(j * 16, 16)] = sd_b[1, pl.ds(j * 16, 16)] + coff
        pltpu.async_copy(hcat_hbm.at[sd_b.at[0]], rows_b, semg)
        pltpu.async_copy(hs_sh.at[sd_b.at[0]], hsg_b, semh)
        pltpu.async_copy(hd_sh.at[doff_b], hdg_b, semd)

    def _wait_scatter(buf):
        sd_b, doff_b, hsg_b, hdg_b, ex_b, rows_b, semg, semh, semd, sems, semx = buf
        pltpu.make_async_copy(ex_b, s_sh.at[sd_b.at[1]], semx).wait()
        pltpu.make_async_copy(rows_b, out_sh.at[sd_b.at[1]], sems).wait()

    def _process(ci, buf):
        sd_b, doff_b, hsg_b, hdg_b, ex_b, rows_b, semg, semh, semd, sems, semx = buf
        base = s * T + ci * B
        pltpu.make_async_copy(hs_sh.at[sd_b.at[0]], hsg_b, semh).wait()
        pltpu.make_async_copy(hd_sh.at[doff_b], hdg_b, semd).wait()
        for j in range(B // 16):
            t = hsg_b[pl.ds(j * 16, 16)] + hdg_b[pl.ds(j * 16, 16)]
            e = jnp.where(t >= 0.0, t, 0.2 * t)
            eid = base + j * 16 + lax.iota(jnp.int32, 16)
            e = jnp.where(eid < EE, e, -1e30)
            ex_b[pl.ds(j * 16, 16)] = jnp.exp(e - g16)
        pltpu.async_copy(ex_b, s_sh.at[sd_b.at[1]], semx, add=True)
        pltpu.make_async_copy(hcat_hbm.at[sd_b.at[0]], rows_b, semg).wait()

        def _scale(r, inner):
            for u in range(4):
                exr = plsc.load_gather(
                    ex_b, (jnp.broadcast_to(4 * r + u, (16,)).astype(jnp.int32),))
                for k in range(D // 16):
                    rows_b[4 * r + u, pl.ds(k * 16, 16)] = (
                        rows_b[4 * r + u, pl.ds(k * 16, 16)] * exr)
            return inner

        lax.fori_loop(0, B // 4, _scale, 0)
        pltpu.async_copy(rows_b, out_sh.at[sd_b.at[1]], sems, add=True)

    _prefetch(0, bufs[0])
    _prefetch(1, bufs[1])
    _process(0, bufs[0])
    _prefetch(2, bufs[2])
    _process(1, bufs[1])

    def _outer(gi, carry):
        for u in range(3):
            ci = 2 + 3 * gi + u
            bn = u
            _wait_scatter(bufs[bn])
            _prefetch(ci + 1, bufs[bn])
            _process(ci, bufs[(2 + u) % 3])
        return carry

    lax.fori_loop(0, (NCHUNK - 3) // 3, _outer, 0)
    _wait_scatter(bufs[NCHUNK % 3])
    _process(NCHUNK - 1, bufs[(NCHUNK - 1) % 3])
    _wait_scatter(bufs[(NCHUNK - 2) % 3])
    _wait_scatter(bufs[(NCHUNK - 1) % 3])
    plsc.subcore_barrier()

    pltpu.sync_copy(out_sh.at[pl.ds(s * NTS, NTS)],
                    out_hbm.at[c, pl.ds(s * NTS, NTS)])
    pltpu.sync_copy(s_sh.at[pl.ds(s * NTS, NTS)],
                    ssum_hbm.at[pl.ds(c * NP + s * NTS, NTS)])


def _buf_types():
    t = []
    for _ in range(3):
        t.append((
            pltpu.VMEM((2, B), jnp.int32),
            pltpu.VMEM((B,), jnp.int32),
            pltpu.VMEM((B,), jnp.float32),
            pltpu.VMEM((B,), jnp.float32),
            pltpu.VMEM((B,), jnp.float32),
            pltpu.VMEM((B, D), jnp.float32),
            pltpu.SemaphoreType.DMA,
            pltpu.SemaphoreType.DMA,
            pltpu.SemaphoreType.DMA,
            pltpu.SemaphoreType.DMA,
            pltpu.SemaphoreType.DMA,
        ))
    return tuple(t)


@functools.partial(
    pl.kernel,
    out_type=[
        jax.ShapeDtypeStruct((2, NP, D), jnp.float32),
        jax.ShapeDtypeStruct((2 * NP,), jnp.float32),
    ],
    mesh=plsc.VectorSubcoreMesh(core_axis_name="c", subcore_axis_name="s"),
    compiler_params=pltpu.CompilerParams(needs_layout_passes=False),
    scratch_types=[
        pltpu.VMEM((16,), jnp.float32),
        pltpu.VMEM((NTS,), jnp.float32),
        _buf_types(),
        pltpu.VMEM_SHARED((NP, D), jnp.float32),
        pltpu.VMEM_SHARED((NP,), jnp.float32),
        pltpu.VMEM_SHARED((2 * NP,), jnp.float32),
        pltpu.VMEM_SHARED((2 * NP,), jnp.float32),
    ],
)
def _aggregate(eidx, hscat, hdcat, g, hcat, out, ssum, *scratch):
    _agg_body(eidx, hscat, hdcat, g, hcat, out, ssum, *scratch)


def _fuse2_body(oi_ref, si_ref, op_ref, sp_ref, b1i_ref, b1p_ref, fha_ref,
                w2i_ref, w2p_ref, asi_ref, adi_ref, asp_ref, adp_ref,
                h2i_ref, h2p_ref, hs2i_ref, hd2i_ref, hs2p_ref, hd2p_ref,
                gsi_ref, gdi_ref, gsp_ref, gdp_ref):
    i = pl.program_id(0)
    xi = jax.nn.relu(oi_ref[...] / (si_ref[...] + EPS) + b1i_ref[...])
    xp = jax.nn.relu(op_ref[...] / (sp_ref[...] + EPS) + b1p_ref[...])
    fha = fha_ref[...]
    wi = jnp.sum(xi * fha[0:1, :], axis=1, keepdims=True)
    wp = jnp.sum(xp * fha[1:2, :], axis=1, keepdims=True)
    mx = jnp.maximum(wi, wp)
    e0 = jnp.exp(wi - mx)
    e1 = jnp.exp(wp - mx)
    xf = (e0 * xi + e1 * xp) / (e0 + e1)
    h2i = jnp.dot(xf, w2i_ref[...], preferred_element_type=jnp.float32)
    h2p = jnp.dot(xf, w2p_ref[...], preferred_element_type=jnp.float32)
    h2i_ref[...] = h2i
    h2p_ref[...] = h2p
    hs2i = jnp.sum(h2i * asi_ref[...], axis=1)
    hd2i = jnp.sum(h2i * adi_ref[...], axis=1)
    hs2p = jnp.sum(h2p * asp_ref[...], axis=1)
    hd2p = jnp.sum(h2p * adp_ref[...], axis=1)
    hs2i_ref[...] = hs2i.reshape(1, BN)
    hd2i_ref[...] = hd2i.reshape(1, BN)
    hs2p_ref[...] = hs2p.reshape(1, BN)
    hd2p_ref[...] = hd2p.reshape(1, BN)

    @pl.when(i == 0)
    def _():
        gsi_ref[0, 0] = jnp.max(hs2i)
        gdi_ref[0, 0] = jnp.max(hd2i)
        gsp_ref[0, 0] = jnp.max(hs2p)
        gdp_ref[0, 0] = jnp.max(hd2p)

    @pl.when(i > 0)
    def _():
        gsi_ref[0, 0] = jnp.maximum(gsi_ref[0, 0], jnp.max(hs2i))
        gdi_ref[0, 0] = jnp.maximum(gdi_ref[0, 0], jnp.max(hd2i))
        gsp_ref[0, 0] = jnp.maximum(gsp_ref[0, 0], jnp.max(hs2p))
        gdp_ref[0, 0] = jnp.maximum(gdp_ref[0, 0], jnp.max(hd2p))


def _fuse2(oi, si, op, sp, b1i, b1p, fha, w2i, w2p, asi, adi, asp, adp):
    grid = (NP // BN,)
    row = lambda i: (i, 0)
    full = lambda i: (0, 0)
    lane = lambda i: (0, i)
    return pl.pallas_call(
        _fuse2_body,
        grid=grid,
        in_specs=[
            pl.BlockSpec((BN, D), row),
            pl.BlockSpec((BN, 1), row),
            pl.BlockSpec((BN, D), row),
            pl.BlockSpec((BN, 1), row),
            pl.BlockSpec((1, D), full),
            pl.BlockSpec((1, D), full),
            pl.BlockSpec((2, D), full),
            pl.BlockSpec((D, D), full),
            pl.BlockSpec((D, D), full),
            pl.BlockSpec((1, D), full),
            pl.BlockSpec((1, D), full),
            pl.BlockSpec((1, D), full),
            pl.BlockSpec((1, D), full),
        ],
        out_specs=[
            pl.BlockSpec((BN, D), row),
            pl.BlockSpec((BN, D), row),
            pl.BlockSpec((1, BN), lane),
            pl.BlockSpec((1, BN), lane),
            pl.BlockSpec((1, BN), lane),
            pl.BlockSpec((1, BN), lane),
            pl.BlockSpec(memory_space=pltpu.SMEM),
            pl.BlockSpec(memory_space=pltpu.SMEM),
            pl.BlockSpec(memory_space=pltpu.SMEM),
            pl.BlockSpec(memory_space=pltpu.SMEM),
        ],
        out_shape=[
            jax.ShapeDtypeStruct((NP, D), jnp.float32),
            jax.ShapeDtypeStruct((NP, D), jnp.float32),
            jax.ShapeDtypeStruct((1, NP), jnp.float32),
            jax.ShapeDtypeStruct((1, NP), jnp.float32),
            jax.ShapeDtypeStruct((1, NP), jnp.float32),
            jax.ShapeDtypeStruct((1, NP), jnp.float32),
            jax.ShapeDtypeStruct((1, 1), jnp.float32),
            jax.ShapeDtypeStruct((1, 1), jnp.float32),
            jax.ShapeDtypeStruct((1, 1), jnp.float32),
            jax.ShapeDtypeStruct((1, 1), jnp.float32),
        ],
    )(oi, si, op, sp, b1i, b1p, fha, w2i, w2p, asi, adi, asp, adp)


def _head_body(oi_ref, si_ref, op_ref, sp_ref, b2i_ref, b2p_ref, aw_ref,
               mw_ref, mb_ref, y_ref):
    xi = oi_ref[...] / (si_ref[...] + EPS) + b2i_ref[...]
    xp = op_ref[...] / (sp_ref[...] + EPS) + b2p_ref[...]
    aw = aw_ref[...]
    wi = jnp.sum(xi * aw[0:1, :], axis=1, keepdims=True)
    wp = jnp.sum(xp * aw[1:2, :], axis=1, keepdims=True)
    mx = jnp.maximum(wi, wp)
    e0 = jnp.exp(wi - mx)
    e1 = jnp.exp(wp - mx)
    x = (e0 * xi + e1 * xp) / (e0 + e1)
    y_ref[...] = jnp.dot(x, mw_ref[...], preferred_element_type=jnp.float32) + mb_ref[...]


def _head(oi, si, op, sp, b2i, b2p, aw, mw, mb):
    grid = (NP // BN,)
    row = lambda i: (i, 0)
    full = lambda i: (0, 0)
    return pl.pallas_call(
        _head_body,
        grid=grid,
        in_specs=[
            pl.BlockSpec((BN, D), row),
            pl.BlockSpec((BN, 1), row),
            pl.BlockSpec((BN, D), row),
            pl.BlockSpec((BN, 1), row),
            pl.BlockSpec((1, D), full),
            pl.BlockSpec((1, D), full),
            pl.BlockSpec((2, D), full),
            pl.BlockSpec((D, 1), full),
            pl.BlockSpec((1, 1), full),
        ],
        out_specs=pl.BlockSpec((BN, 1), row),
        out_shape=jax.ShapeDtypeStruct((NP, 1), jnp.float32),
    )(oi, si, op, sp, b2i, b2p, aw, mw, mb)


def kernel(x_industry, edge_index_industry, x_pos_corr, edge_index_pos,
           W1i, a1i_s, a1i_d, b1i, W2i, a2i_s, a2i_d, b2i,
           W1p, a1p_s, a1p_d, b1p, W2p, a2p_s, a2p_d, b2p,
           fha, aw, mlp_W, mlp_b):
    f32 = jnp.float32
    loop = jnp.arange(N, dtype=jnp.int32)
    zpad = jnp.zeros((EEP - EE,), jnp.int32)

    def edges(ei, off):
        src = jnp.concatenate([ei[0].astype(jnp.int32), loop, zpad]) + off
        dst = jnp.concatenate([ei[1].astype(jnp.int32), loop, zpad])
        return src, dst

    src_i, dst_i = edges(edge_index_industry, 0)
    src_p, dst_p = edges(edge_index_pos, NP)
    srcoff = jnp.concatenate([src_i, src_p]).reshape(2 * NTILES * NCHUNK, B)
    dstloc = jnp.concatenate([dst_i, dst_p]).reshape(2 * NTILES * NCHUNK, B)
    eidx = jnp.stack([srcoff, dstloc], axis=1)

    xpad = jnp.zeros((NP - N, D), f32)
    xi_in = jnp.concatenate([x_industry, xpad], axis=0)
    xp_in = jnp.concatenate([x_pos_corr, xpad], axis=0)

    r1 = lambda v: v.reshape(1, D)

    h1i, hs1i, hd1i, gs1i, gd1i = _dense1(xi_in, W1i, r1(a1i_s), r1(a1i_d))
    h1p, hs1p, hd1p, gs1p, gd1p = _dense1(xp_in, W1p, r1(a1p_s), r1(a1p_d))

    def gbound(gs, gd):
        t = gs[0, 0] + gd[0, 0]
        return jnp.broadcast_to(jnp.where(t >= 0.0, t, 0.2 * t), (16,))

    g1 = jnp.concatenate([gbound(gs1i, gd1i), gbound(gs1p, gd1p)])
    hscat1 = jnp.concatenate([hs1i[0], hs1p[0]])
    hdcat1 = jnp.concatenate([hd1i[0], hd1p[0]])
    hcat1 = jnp.concatenate([h1i, h1p], axis=0)

    out1, ssum1 = _aggregate(eidx, hscat1, hdcat1, g1, hcat1)

    c1 = lambda v: v.reshape(NP, 1)
    h2i, h2p, hs2i, hd2i, hs2p, hd2p, gs2i, gd2i, gs2p, gd2p = _fuse2(
        out1[0], c1(ssum1[:NP]), out1[1], c1(ssum1[NP:]),
        r1(b1i), r1(b1p), fha, W2i, W2p,
        r1(a2i_s), r1(a2i_d), r1(a2p_s), r1(a2p_d))

    g2 = jnp.concatenate([gbound(gs2i, gd2i), gbound(gs2p, gd2p)])
    hscat2 = jnp.concatenate([hs2i[0], hs2p[0]])
    hdcat2 = jnp.concatenate([hd2i[0], hd2p[0]])
    hcat2 = jnp.concatenate([h2i, h2p], axis=0)

    out2, ssum2 = _aggregate(eidx, hscat2, hdcat2, g2, hcat2)

    y = _head(out2[0], c1(ssum2[:NP]), out2[1], c1(ssum2[NP:]),
              r1(b2i), r1(b2p), aw, mlp_W, mlp_b.reshape(1, 1))
    return y[:N]


# speedup vs baseline: 41.6163x; 1.0022x over previous
import functools

import jax
import jax.numpy as jnp
from jax import lax
from jax.experimental import pallas as pl
from jax.experimental.pallas import tpu as pltpu
from jax.experimental.pallas import tpu_sc as plsc

N = 10000
NP = 10240
D = 128
EE = 330000
NTILES = 16
T = 20736
B = 96
NCHUNK = T // B
EEP = NTILES * T
NTS = NP // NTILES
BN = 1024
EPS = 1e-16


def _dense_body(x_ref, w_ref, as_ref, ad_ref, h_ref, hs_ref, hd_ref, gs_ref, gd_ref):
    g = pl.program_id(0)
    i = pl.program_id(1)
    h = jnp.dot(x_ref[0], w_ref[0], preferred_element_type=jnp.float32)
    h_ref[0] = h
    hs = jnp.sum(h * as_ref[0], axis=1)
    hd = jnp.sum(h * ad_ref[0], axis=1)
    hs_ref[...] = hs.reshape(1, 1, BN)
    hd_ref[...] = hd.reshape(1, 1, BN)
    bs = jnp.max(hs)
    bd = jnp.max(hd)

    @pl.when(i == 0)
    def _():
        gs_ref[g, 0] = bs
        gd_ref[g, 0] = bd

    @pl.when(i > 0)
    def _():
        gs_ref[g, 0] = jnp.maximum(gs_ref[g, 0], bs)
        gd_ref[g, 0] = jnp.maximum(gd_ref[g, 0], bd)


def _dense1(x2, w2, as2, ad2):
    grid = (2, NP // BN)
    return pl.pallas_call(
        _dense_body,
        grid=grid,
        in_specs=[
            pl.BlockSpec((1, BN, D), lambda g, i: (g, i, 0)),
            pl.BlockSpec((1, D, D), lambda g, i: (g, 0, 0)),
            pl.BlockSpec((1, 1, D), lambda g, i: (g, 0, 0)),
            pl.BlockSpec((1, 1, D), lambda g, i: (g, 0, 0)),
        ],
        out_specs=[
            pl.BlockSpec((1, BN, D), lambda g, i: (g, i, 0)),
            pl.BlockSpec((1, 1, BN), lambda g, i: (g, 0, i)),
            pl.BlockSpec((1, 1, BN), lambda g, i: (g, 0, i)),
            pl.BlockSpec(memory_space=pltpu.SMEM),
            pl.BlockSpec(memory_space=pltpu.SMEM),
        ],
        out_shape=[
            jax.ShapeDtypeStruct((2, NP, D), jnp.float32),
            jax.ShapeDtypeStruct((2, 1, NP), jnp.float32),
            jax.ShapeDtypeStruct((2, 1, NP), jnp.float32),
            jax.ShapeDtypeStruct((2, 1), jnp.float32),
            jax.ShapeDtypeStruct((2, 1), jnp.float32),
        ],
    )(x2, w2, as2, ad2)


def _agg_body(eidx_hbm, hs_hbm, hd_hbm, g_hbm, hcat_hbm,
              out_hbm, ssum_hbm,
              g_v, zs_v, bufs, out_sh, s_sh, hs_sh, hd_sh):
    c = lax.axis_index("c")
    s = lax.axis_index("s")

    pltpu.sync_copy(g_hbm.at[pl.ds(c * 16, 16)], g_v)
    g16 = g_v[...]
    coff = (c * NP).astype(jnp.int32)

    @pl.when(s == 0)
    def _():
        pltpu.sync_copy(hs_hbm, hs_sh)
        pltpu.sync_copy(hd_hbm, hd_sh)

    z16 = jnp.zeros((16,), jnp.float32)
    rows0 = bufs[0][5]

    def _zrow(r, carry):
        for k in range(D // 16):
            rows0[r, pl.ds(k * 16, 16)] = z16
        return carry

    lax.fori_loop(0, B, _zrow, 0)
    for k in range(NTS // 16):
        zs_v[pl.ds(k * 16, 16)] = z16
    for j in range(NTS // B):
        pltpu.sync_copy(rows0, out_sh.at[pl.ds(s * NTS + j * B, B)])
    rem = NTS - (NTS // B) * B
    if rem:
        pltpu.sync_copy(rows0.at[pl.ds(0, rem)],
                        out_sh.at[pl.ds(s * NTS + (NTS // B) * B, rem)])
    pltpu.sync_copy(zs_v, s_sh.at[pl.ds(s * NTS, NTS)])
    plsc.subcore_barrier()

    def _prefetch(ci, buf):
        sd_b, doff_b, hsg_b, hdg_b, ex_b, rows_b, semg, semh, semd, sems, semx = buf
        rowi = (c * NTILES + s) * NCHUNK + ci
        pltpu.sync_copy(eidx_hbm.at[rowi], sd_b)
        for j in range(B // 16):
            doff_b[pl.ds(j * 16, 16)] = sd_b[1, pl.ds(j * 16, 16)] + coff
        pltpu.async_copy(hcat_hbm.at[sd_b.at[0]], rows_b, semg)
        pltpu.async_copy(hs_sh.at[sd_b.at[0]], hsg_b, semh)
        pltpu.async_copy(hd_sh.at[doff_b], hdg_b, semd)

    def _wait_scatter(buf):
        sd_b, doff_b, hsg_b, hdg_b, ex_b, rows_b, semg, semh, semd, sems, semx = buf
        pltpu.make_async_copy(ex_b, s_sh.at[sd_b.at[1]], semx).wait()
        pltpu.make_async_copy(rows_b, out_sh.at[sd_b.at[1]], sems).wait()

    def _process(ci, buf):
        sd_b, doff_b, hsg_b, hdg_b, ex_b, rows_b, semg, semh, semd, sems, semx = buf
        base = s * T + ci * B
        pltpu.make_async_copy(hs_sh.at[sd_b.at[0]], hsg_b, semh).wait()
        pltpu.make_async_copy(hd_sh.at[doff_b], hdg_b, semd).wait()
        for j in range(B // 16):
            t = hsg_b[pl.ds(j * 16, 16)] + hdg_b[pl.ds(j * 16, 16)]
            e = jnp.where(t >= 0.0, t, 0.2 * t)
            eid = base + j * 16 + lax.iota(jnp.int32, 16)
            e = jnp.where(eid < EE, e, -1e30)
            ex_b[pl.ds(j * 16, 16)] = jnp.exp(e - g16)
        pltpu.async_copy(ex_b, s_sh.at[sd_b.at[1]], semx, add=True)
        pltpu.make_async_copy(hcat_hbm.at[sd_b.at[0]], rows_b, semg).wait()

        def _scale(r, inner):
            for u in range(4):
                exr = plsc.load_gather(
                    ex_b, (jnp.broadcast_to(4 * r + u, (16,)).astype(jnp.int32),))
                for k in range(D // 16):
                    rows_b[4 * r + u, pl.ds(k * 16, 16)] = (
                        rows_b[4 * r + u, pl.ds(k * 16, 16)] * exr)
            return inner

        lax.fori_loop(0, B // 4, _scale, 0)
        pltpu.async_copy(rows_b, out_sh.at[sd_b.at[1]], sems, add=True)

    _prefetch(0, bufs[0])
    _prefetch(1, bufs[1])
    _process(0, bufs[0])
    _prefetch(2, bufs[2])
    _process(1, bufs[1])

    def _outer(gi, carry):
        for u in range(3):
            ci = 2 + 3 * gi + u
            bn = u
            _wait_scatter(bufs[bn])
            _prefetch(ci + 1, bufs[bn])
            _process(ci, bufs[(2 + u) % 3])
        return carry

    lax.fori_loop(0, (NCHUNK - 3) // 3, _outer, 0)
    _wait_scatter(bufs[NCHUNK % 3])
    _process(NCHUNK - 1, bufs[(NCHUNK - 1) % 3])
    _wait_scatter(bufs[(NCHUNK - 2) % 3])
    _wait_scatter(bufs[(NCHUNK - 1) % 3])
    plsc.subcore_barrier()

    pltpu.sync_copy(out_sh.at[pl.ds(s * NTS, NTS)],
                    out_hbm.at[c, pl.ds(s * NTS, NTS)])
    pltpu.sync_copy(s_sh.at[pl.ds(s * NTS, NTS)],
                    ssum_hbm.at[pl.ds(c * NP + s * NTS, NTS)])


def _buf_types():
    t = []
    for _ in range(3):
        t.append((
            pltpu.VMEM((2, B), jnp.int32),
            pltpu.VMEM((B,), jnp.int32),
            pltpu.VMEM((B,), jnp.float32),
            pltpu.VMEM((B,), jnp.float32),
            pltpu.VMEM((B,), jnp.float32),
            pltpu.VMEM((B, D), jnp.float32),
            pltpu.SemaphoreType.DMA,
            pltpu.SemaphoreType.DMA,
            pltpu.SemaphoreType.DMA,
            pltpu.SemaphoreType.DMA,
            pltpu.SemaphoreType.DMA,
        ))
    return tuple(t)


@functools.partial(
    pl.kernel,
    out_type=[
        jax.ShapeDtypeStruct((2, NP, D), jnp.float32),
        jax.ShapeDtypeStruct((2 * NP,), jnp.float32),
    ],
    mesh=plsc.VectorSubcoreMesh(core_axis_name="c", subcore_axis_name="s"),
    compiler_params=pltpu.CompilerParams(needs_layout_passes=False),
    scratch_types=[
        pltpu.VMEM((16,), jnp.float32),
        pltpu.VMEM((NTS,), jnp.float32),
        _buf_types(),
        pltpu.VMEM_SHARED((NP, D), jnp.float32),
        pltpu.VMEM_SHARED((NP,), jnp.float32),
        pltpu.VMEM_SHARED((2 * NP,), jnp.float32),
        pltpu.VMEM_SHARED((2 * NP,), jnp.float32),
    ],
)
def _aggregate(eidx, hscat, hdcat, g, hcat, out, ssum, *scratch):
    _agg_body(eidx, hscat, hdcat, g, hcat, out, ssum, *scratch)


def _fuse2_body(oi_ref, si_ref, op_ref, sp_ref, b1i_ref, b1p_ref, fha_ref,
                w2i_ref, w2p_ref, asi_ref, adi_ref, asp_ref, adp_ref,
                h2i_ref, h2p_ref, hs2i_ref, hd2i_ref, hs2p_ref, hd2p_ref,
                gsi_ref, gdi_ref, gsp_ref, gdp_ref):
    i = pl.program_id(0)
    xi = jax.nn.relu(oi_ref[...] / (si_ref[...] + EPS) + b1i_ref[...])
    xp = jax.nn.relu(op_ref[...] / (sp_ref[...] + EPS) + b1p_ref[...])
    fha = fha_ref[...]
    wi = jnp.sum(xi * fha[0:1, :], axis=1, keepdims=True)
    wp = jnp.sum(xp * fha[1:2, :], axis=1, keepdims=True)
    mx = jnp.maximum(wi, wp)
    e0 = jnp.exp(wi - mx)
    e1 = jnp.exp(wp - mx)
    xf = (e0 * xi + e1 * xp) / (e0 + e1)
    h2i = jnp.dot(xf, w2i_ref[...], preferred_element_type=jnp.float32)
    h2p = jnp.dot(xf, w2p_ref[...], preferred_element_type=jnp.float32)
    h2i_ref[...] = h2i
    h2p_ref[...] = h2p
    hs2i = jnp.sum(h2i * asi_ref[...], axis=1)
    hd2i = jnp.sum(h2i * adi_ref[...], axis=1)
    hs2p = jnp.sum(h2p * asp_ref[...], axis=1)
    hd2p = jnp.sum(h2p * adp_ref[...], axis=1)
    hs2i_ref[...] = hs2i.reshape(1, BN)
    hd2i_ref[...] = hd2i.reshape(1, BN)
    hs2p_ref[...] = hs2p.reshape(1, BN)
    hd2p_ref[...] = hd2p.reshape(1, BN)

    @pl.when(i == 0)
    def _():
        gsi_ref[0, 0] = jnp.max(hs2i)
        gdi_ref[0, 0] = jnp.max(hd2i)
        gsp_ref[0, 0] = jnp.max(hs2p)
        gdp_ref[0, 0] = jnp.max(hd2p)

    @pl.when(i > 0)
    def _():
        gsi_ref[0, 0] = jnp.maximum(gsi_ref[0, 0], jnp.max(hs2i))
        gdi_ref[0, 0] = jnp.maximum(gdi_ref[0, 0], jnp.max(hd2i))
        gsp_ref[0, 0] = jnp.maximum(gsp_ref[0, 0], jnp.max(hs2p))
        gdp_ref[0, 0] = jnp.maximum(gdp_ref[0, 0], jnp.max(hd2p))


def _fuse2(oi, si, op, sp, b1i, b1p, fha, w2i, w2p, asi, adi, asp, adp):
    grid = (NP // BN,)
    row = lambda i: (i, 0)
    full = lambda i: (0, 0)
    lane = lambda i: (0, i)
    return pl.pallas_call(
        _fuse2_body,
        grid=grid,
        in_specs=[
            pl.BlockSpec((BN, D), row),
            pl.BlockSpec((BN, 1), row),
            pl.BlockSpec((BN, D), row),
            pl.BlockSpec((BN, 1), row),
            pl.BlockSpec((1, D), full),
            pl.BlockSpec((1, D), full),
            pl.BlockSpec((2, D), full),
            pl.BlockSpec((D, D), full),
            pl.BlockSpec((D, D), full),
            pl.BlockSpec((1, D), full),
            pl.BlockSpec((1, D), full),
            pl.BlockSpec((1, D), full),
            pl.BlockSpec((1, D), full),
        ],
        out_specs=[
            pl.BlockSpec((BN, D), row),
            pl.BlockSpec((BN, D), row),
            pl.BlockSpec((1, BN), lane),
            pl.BlockSpec((1, BN), lane),
            pl.BlockSpec((1, BN), lane),
            pl.BlockSpec((1, BN), lane),
            pl.BlockSpec(memory_space=pltpu.SMEM),
            pl.BlockSpec(memory_space=pltpu.SMEM),
            pl.BlockSpec(memory_space=pltpu.SMEM),
            pl.BlockSpec(memory_space=pltpu.SMEM),
        ],
        out_shape=[
            jax.ShapeDtypeStruct((NP, D), jnp.float32),
            jax.ShapeDtypeStruct((NP, D), jnp.float32),
            jax.ShapeDtypeStruct((1, NP), jnp.float32),
            jax.ShapeDtypeStruct((1, NP), jnp.float32),
            jax.ShapeDtypeStruct((1, NP), jnp.float32),
            jax.ShapeDtypeStruct((1, NP), jnp.float32),
            jax.ShapeDtypeStruct((1, 1), jnp.float32),
            jax.ShapeDtypeStruct((1, 1), jnp.float32),
            jax.ShapeDtypeStruct((1, 1), jnp.float32),
            jax.ShapeDtypeStruct((1, 1), jnp.float32),
        ],
    )(oi, si, op, sp, b1i, b1p, fha, w2i, w2p, asi, adi, asp, adp)


def _head_body(oi_ref, si_ref, op_ref, sp_ref, b2i_ref, b2p_ref, aw_ref,
               mw_ref, mb_ref, y_ref):
    xi = oi_ref[...] / (si_ref[...] + EPS) + b2i_ref[...]
    xp = op_ref[...] / (sp_ref[...] + EPS) + b2p_ref[...]
    aw = aw_ref[...]
    wi = jnp.sum(xi * aw[0:1, :], axis=1, keepdims=True)
    wp = jnp.sum(xp * aw[1:2, :], axis=1, keepdims=True)
    mx = jnp.maximum(wi, wp)
    e0 = jnp.exp(wi - mx)
    e1 = jnp.exp(wp - mx)
    x = (e0 * xi + e1 * xp) / (e0 + e1)
    y_ref[...] = jnp.dot(x, mw_ref[...], preferred_element_type=jnp.float32) + mb_ref[...]


def _head(oi, si, op, sp, b2i, b2p, aw, mw, mb):
    grid = (NP // BN,)
    row = lambda i: (i, 0)
    full = lambda i: (0, 0)
    return pl.pallas_call(
        _head_body,
        grid=grid,
        in_specs=[
            pl.BlockSpec((BN, D), row),
            pl.BlockSpec((BN, 1), row),
            pl.BlockSpec((BN, D), row),
            pl.BlockSpec((BN, 1), row),
            pl.BlockSpec((1, D), full),
            pl.BlockSpec((1, D), full),
            pl.BlockSpec((2, D), full),
            pl.BlockSpec((D, 1), full),
            pl.BlockSpec((1, 1), full),
        ],
        out_specs=pl.BlockSpec((BN, 1), row),
        out_shape=jax.ShapeDtypeStruct((NP, 1), jnp.float32),
    )(oi, si, op, sp, b2i, b2p, aw, mw, mb)


def kernel(x_industry, edge_index_industry, x_pos_corr, edge_index_pos,
           W1i, a1i_s, a1i_d, b1i, W2i, a2i_s, a2i_d, b2i,
           W1p, a1p_s, a1p_d, b1p, W2p, a2p_s, a2p_d, b2p,
           fha, aw, mlp_W, mlp_b):
    f32 = jnp.float32
    loop = jnp.arange(N, dtype=jnp.int32)
    zpad = jnp.zeros((EEP - EE,), jnp.int32)

    def edges(ei, off):
        src = jnp.concatenate([ei[0].astype(jnp.int32), loop, zpad]) + off
        dst = jnp.concatenate([ei[1].astype(jnp.int32), loop, zpad])
        return src, dst

    src_i, dst_i = edges(edge_index_industry, 0)
    src_p, dst_p = edges(edge_index_pos, NP)
    srcoff = jnp.concatenate([src_i, src_p]).reshape(2 * NTILES * NCHUNK, B)
    dstloc = jnp.concatenate([dst_i, dst_p]).reshape(2 * NTILES * NCHUNK, B)
    eidx = jnp.stack([srcoff, dstloc], axis=1)

    xpad = jnp.zeros((NP - N, D), f32)
    xi_in = jnp.concatenate([x_industry, xpad], axis=0)
    xp_in = jnp.concatenate([x_pos_corr, xpad], axis=0)
    x2 = jnp.stack([xi_in, xp_in])
    w2s = jnp.stack([W1i, W1p])
    as2 = jnp.stack([a1i_s.reshape(1, D), a1p_s.reshape(1, D)])
    ad2 = jnp.stack([a1i_d.reshape(1, D), a1p_d.reshape(1, D)])

    r1 = lambda v: v.reshape(1, D)

    h1, hs1, hd1, gs1, gd1 = _dense1(x2, w2s, as2, ad2)

    def gbound(gs, gd):
        t = gs + gd
        return jnp.broadcast_to(jnp.where(t >= 0.0, t, 0.2 * t), (16,))

    g1 = jnp.concatenate([gbound(gs1[0, 0], gd1[0, 0]),
                          gbound(gs1[1, 0], gd1[1, 0])])
    hscat1 = hs1.reshape(2 * NP)
    hdcat1 = hd1.reshape(2 * NP)
    hcat1 = h1.reshape(2 * NP, D)

    out1, ssum1 = _aggregate(eidx, hscat1, hdcat1, g1, hcat1)

    c1 = lambda v: v.reshape(NP, 1)
    h2i, h2p, hs2i, hd2i, hs2p, hd2p, gs2i, gd2i, gs2p, gd2p = _fuse2(
        out1[0], c1(ssum1[:NP]), out1[1], c1(ssum1[NP:]),
        r1(b1i), r1(b1p), fha, W2i, W2p,
        r1(a2i_s), r1(a2i_d), r1(a2p_s), r1(a2p_d))

    g2 = jnp.concatenate([gbound(gs2i[0, 0], gd2i[0, 0]),
                          gbound(gs2p[0, 0], gd2p[0, 0])])
    hscat2 = jnp.concatenate([hs2i[0], hs2p[0]])
    hdcat2 = jnp.concatenate([hd2i[0], hd2p[0]])
    hcat2 = jnp.concatenate([h2i, h2p], axis=0)

    out2, ssum2 = _aggregate(eidx, hscat2, hdcat2, g2, hcat2)

    y = _head(out2[0], c1(ssum2[:NP]), out2[1], c1(ssum2[NP:]),
              r1(b2i), r1(b2p), aw, mlp_W, mlp_b.reshape(1, 1))
    return y[:N]
